# Initial kernel scaffold; baseline (speedup 1.0000x reference)
import jax
import jax.numpy as jnp
from jax.experimental import pallas as pl


def kernel(node_features, edge_index, edge_attr, msg_w1, msg_b1, msg_w2, msg_b2, q_w, q_b, kv_w, kv_b, out_w1, out_b1, out_w2, out_b2, ln_g, ln_b):
    raise NotImplementedError("write your pallas kernel here")


# speedup vs baseline: 1.3964x; 1.3964x over previous
import functools
import jax
import jax.numpy as jnp
from jax.experimental import pallas as pl

N = 10000
E = 320000
DIN = 128
DOUT = 128
H = 8
HD = DOUT // H
DE = 16


def _erf(x):
    p = 0.3275911
    a1, a2, a3, a4, a5 = (0.254829592, -0.284496736, 1.421413741,
                          -1.453152027, 1.061405429)
    ax = jnp.abs(x)
    t = 1.0 / (1.0 + p * ax)
    poly = ((((a5 * t + a4) * t + a3) * t + a2) * t + a1) * t
    y = 1.0 - poly * jnp.exp(-ax * ax)
    return jnp.sign(x) * y


def _gelu(x):
    return 0.5 * x * (1.0 + _erf(x * 0.7071067811865476))


def _proj_body(x_ref, w_ref, o_ref):
    o_ref[...] = jnp.dot(x_ref[...], w_ref[...],
                         preferred_element_type=jnp.float32)


def _node_proj(x, w_cat):
    bn = 2000
    return pl.pallas_call(
        _proj_body,
        grid=(N // bn,),
        in_specs=[pl.BlockSpec((bn, DIN), lambda i: (i, 0)),
                  pl.BlockSpec((DIN, 4 * DOUT), lambda i: (0, 0))],
        out_specs=pl.BlockSpec((bn, 4 * DOUT), lambda i: (i, 0)),
        out_shape=jax.ShapeDtypeStruct((N, 4 * DOUT), jnp.float32),
    )(x, w_cat)


def _edge_body(s1_ref, ea_ref, w1e_ref, b1_ref, o_ref):
    m1 = s1_ref[...] + jnp.dot(ea_ref[...], w1e_ref[...],
                               preferred_element_type=jnp.float32) + b1_ref[...]
    o_ref[...] = _gelu(m1)


def _edge_gelu(s1, ea, w1e, b1):
    be = 4000
    return pl.pallas_call(
        _edge_body,
        grid=(E // be,),
        in_specs=[pl.BlockSpec((be, DOUT), lambda i: (i, 0)),
                  pl.BlockSpec((be, DE), lambda i: (i, 0)),
                  pl.BlockSpec((DE, DOUT), lambda i: (0, 0)),
                  pl.BlockSpec((1, DOUT), lambda i: (0, 0))],
        out_specs=pl.BlockSpec((be, DOUT), lambda i: (i, 0)),
        out_shape=jax.ShapeDtypeStruct((E, DOUT), jnp.float32),
    )(s1, ea, w1e, b1.reshape(1, DOUT))


def _final_body(x_ref, aggs_ref, cnt_ref, w_ref, msgw2_ref, msgb2_ref,
                w1a_ref, w1b_ref, b1_ref, w2_ref, b2_ref, g_ref, bb_ref, o_ref):
    x = x_ref[...]
    agg = jnp.dot(aggs_ref[...], msgw2_ref[...],
                  preferred_element_type=jnp.float32)
    agg = agg + cnt_ref[...] * msgb2_ref[...]
    scale = jnp.repeat(w_ref[...], HD, axis=1)
    agg = agg * scale
    h = jnp.dot(x, w1a_ref[...], preferred_element_type=jnp.float32)
    h = h + jnp.dot(agg, w1b_ref[...], preferred_element_type=jnp.float32)
    h = _gelu(h + b1_ref[...])
    h = jnp.dot(h, w2_ref[...], preferred_element_type=jnp.float32) + b2_ref[...]
    h = x + h
    mu = jnp.mean(h, axis=-1, keepdims=True)
    var = jnp.mean((h - mu) ** 2, axis=-1, keepdims=True)
    o_ref[...] = (h - mu) * jax.lax.rsqrt(var + 1e-5) * g_ref[...] + bb_ref[...]


def _final(x, aggsum, cnt, w, msg_w2, msg_b2, out_w1, out_b1, out_w2, out_b2,
           ln_g, ln_b):
    bn = 2000
    row = lambda i: (i, 0)
    full = lambda i: (0, 0)
    return pl.pallas_call(
        _final_body,
        grid=(N // bn,),
        in_specs=[pl.BlockSpec((bn, DIN), row),
                  pl.BlockSpec((bn, DOUT), row),
                  pl.BlockSpec((bn, 1), row),
                  pl.BlockSpec((bn, H), row),
                  pl.BlockSpec((DOUT, DOUT), full),
                  pl.BlockSpec((1, DOUT), full),
                  pl.BlockSpec((DIN, DOUT), full),
                  pl.BlockSpec((DOUT, DOUT), full),
                  pl.BlockSpec((1, DOUT), full),
                  pl.BlockSpec((DOUT, DOUT), full),
                  pl.BlockSpec((1, DOUT), full),
                  pl.BlockSpec((1, DOUT), full),
                  pl.BlockSpec((1, DOUT), full)],
        out_specs=pl.BlockSpec((bn, DOUT), row),
        out_shape=jax.ShapeDtypeStruct((N, DOUT), jnp.float32),
    )(x, aggsum, cnt.reshape(N, 1), w, msg_w2, msg_b2.reshape(1, DOUT),
      out_w1[:DIN], out_w1[DIN:], out_b1.reshape(1, DOUT), out_w2,
      out_b2.reshape(1, DOUT), ln_g.reshape(1, DOUT), ln_b.reshape(1, DOUT))


def kernel(node_features, edge_index, edge_attr, msg_w1, msg_b1, msg_w2, msg_b2,
           q_w, q_b, kv_w, kv_b, out_w1, out_b1, out_w2, out_b2, ln_g, ln_b):
    x = node_features
    src, dst = edge_index[0], edge_index[1]

    w_cat = jnp.concatenate([msg_w1[:DIN], msg_w1[DIN:2 * DIN], q_w,
                             kv_w[:DIN]], axis=1)
    P = _node_proj(x, w_cat)
    A, B, Q, K = (P[:, :DOUT], P[:, DOUT:2 * DOUT],
                  P[:, 2 * DOUT:3 * DOUT] + q_b, P[:, 3 * DOUT:])

    s1 = A[src] + B[dst]
    g = _edge_gelu(s1, edge_attr, msg_w1[2 * DIN:], msg_b1)

    aggsum = jnp.zeros((N, DOUT), jnp.float32).at[dst].add(g)
    cnt = jnp.zeros((N,), jnp.float32).at[dst].add(1.0)
    win = jnp.full((N,), -1, jnp.int32).at[dst].max(
        jnp.arange(E, dtype=jnp.int32))

    winc = jnp.maximum(win, 0)
    kvw = K[src[winc]] + edge_attr[winc] @ kv_w[DIN:] + kv_b
    attn = (Q.reshape(N, H, HD) * kvw.reshape(N, H, HD)).sum(-1) * (HD ** -0.5)
    attn = jnp.where(win[:, None] >= 0, attn, -jnp.inf)
    w = jax.nn.softmax(attn, axis=0)

    return _final(x, aggsum, cnt, w, msg_w2, msg_b2, out_w1, out_b1, out_w2,
                  out_b2, ln_g, ln_b)


# speedup vs baseline: 2.1180x; 1.5168x over previous
import functools
import jax
import jax.numpy as jnp
from jax import lax
from jax.experimental import pallas as pl
from jax.experimental.pallas import tpu as pltpu
from jax.experimental.pallas import tpu_sc as plsc

N = 10000
E = 320000
DIN = 128
DOUT = 128
H = 8
HD = DOUT // H
DE = 16

NC = 2
NS = 16
NW = NC * NS
NP = 10240
NSL = NP // NS
EC = E // NW
CB = 80
NCH = EC // CB


def _erf(x):
    p = 0.3275911
    a1, a2, a3, a4, a5 = (0.254829592, -0.284496736, 1.421413741,
                          -1.453152027, 1.061405429)
    ax = jnp.abs(x)
    t = 1.0 / (1.0 + p * ax)
    poly = ((((a5 * t + a4) * t + a3) * t + a2) * t + a1) * t
    y = 1.0 - poly * jnp.exp(-ax * ax)
    return jnp.sign(x) * y


def _gelu(x):
    return 0.5 * x * (1.0 + _erf(x * 0.7071067811865476))


def _proj_body(x_ref, w_ref, o_ref):
    o_ref[...] = jnp.dot(x_ref[...], w_ref[...],
                         preferred_element_type=jnp.float32)


def _node_proj(x, w_cat):
    bn = 2000
    return pl.pallas_call(
        _proj_body,
        grid=(N // bn,),
        in_specs=[pl.BlockSpec((bn, DIN), lambda i: (i, 0)),
                  pl.BlockSpec((DIN, 4 * DOUT), lambda i: (0, 0))],
        out_specs=pl.BlockSpec((bn, 4 * DOUT), lambda i: (i, 0)),
        out_shape=jax.ShapeDtypeStruct((N, 4 * DOUT), jnp.float32),
    )(x, w_cat)


def _edge_body(s1_ref, ea_ref, w1e_ref, b1_ref, o_ref):
    m1 = s1_ref[...] + jnp.dot(ea_ref[...], w1e_ref[...],
                               preferred_element_type=jnp.float32) + b1_ref[...]
    o_ref[...] = _gelu(m1)


def _edge_gelu(s1, ea, w1e, b1):
    be = 4000
    return pl.pallas_call(
        _edge_body,
        grid=(E // be,),
        in_specs=[pl.BlockSpec((be, DOUT), lambda i: (i, 0)),
                  pl.BlockSpec((be, DE), lambda i: (i, 0)),
                  pl.BlockSpec((DE, DOUT), lambda i: (0, 0)),
                  pl.BlockSpec((1, DOUT), lambda i: (0, 0))],
        out_specs=pl.BlockSpec((be, DOUT), lambda i: (i, 0)),
        out_shape=jax.ShapeDtypeStruct((E, DOUT), jnp.float32),
    )(s1, ea, w1e, b1.reshape(1, DOUT))


def _final_body(x_ref, aggs_ref, cnt_ref, w_ref, msgw2_ref, msgb2_ref,
                w1a_ref, w1b_ref, b1_ref, w2_ref, b2_ref, g_ref, bb_ref, o_ref):
    x = x_ref[...]
    agg = jnp.dot(aggs_ref[...], msgw2_ref[...],
                  preferred_element_type=jnp.float32)
    agg = agg + cnt_ref[...] * msgb2_ref[...]
    scale = jnp.repeat(w_ref[...], HD, axis=1)
    agg = agg * scale
    h = jnp.dot(x, w1a_ref[...], preferred_element_type=jnp.float32)
    h = h + jnp.dot(agg, w1b_ref[...], preferred_element_type=jnp.float32)
    h = _gelu(h + b1_ref[...])
    h = jnp.dot(h, w2_ref[...], preferred_element_type=jnp.float32) + b2_ref[...]
    h = x + h
    mu = jnp.mean(h, axis=-1, keepdims=True)
    var = jnp.mean((h - mu) ** 2, axis=-1, keepdims=True)
    o_ref[...] = (h - mu) * jax.lax.rsqrt(var + 1e-5) * g_ref[...] + bb_ref[...]


def _final(x, aggsum, cnt, w, msg_w2, msg_b2, out_w1, out_b1, out_w2, out_b2,
           ln_g, ln_b):
    bn = 2000
    row = lambda i: (i, 0)
    full = lambda i: (0, 0)
    return pl.pallas_call(
        _final_body,
        grid=(N // bn,),
        in_specs=[pl.BlockSpec((bn, DIN), row),
                  pl.BlockSpec((bn, DOUT), row),
                  pl.BlockSpec((bn, 1), row),
                  pl.BlockSpec((bn, H), row),
                  pl.BlockSpec((DOUT, DOUT), full),
                  pl.BlockSpec((1, DOUT), full),
                  pl.BlockSpec((DIN, DOUT), full),
                  pl.BlockSpec((DOUT, DOUT), full),
                  pl.BlockSpec((1, DOUT), full),
                  pl.BlockSpec((DOUT, DOUT), full),
                  pl.BlockSpec((1, DOUT), full),
                  pl.BlockSpec((1, DOUT), full),
                  pl.BlockSpec((1, DOUT), full)],
        out_specs=pl.BlockSpec((bn, DOUT), row),
        out_shape=jax.ShapeDtypeStruct((N, DOUT), jnp.float32),
    )(x, aggsum, cnt.reshape(N, 1), w, msg_w2, msg_b2.reshape(1, DOUT),
      out_w1[:DIN], out_w1[DIN:], out_b1.reshape(1, DOUT), out_w2,
      out_b2.reshape(1, DOUT), ln_g.reshape(1, DOUT), ln_b.reshape(1, DOUT))


_SC_MESH = plsc.VectorSubcoreMesh(core_axis_name="c", subcore_axis_name="s")


@functools.partial(
    pl.kernel,
    mesh=_SC_MESH,
    out_type=[
        jax.ShapeDtypeStruct((NC, NP, DOUT), jnp.float32),
        jax.ShapeDtypeStruct((NW, NP), jnp.float32),
        jax.ShapeDtypeStruct((NC, NP), jnp.int32),
    ],
    scratch_types=[
        pltpu.VMEM((CB,), jnp.int32),
        pltpu.VMEM((CB, DOUT), jnp.float32),
        pltpu.VMEM((NP,), jnp.float32),
        pltpu.VMEM((NP,), jnp.int32),
        pltpu.VMEM((NSL,), jnp.int32),
        pltpu.VMEM((NSL,), jnp.int32),
        pltpu.VMEM_SHARED((NP, DOUT), jnp.float32),
        pltpu.VMEM_SHARED((NS, NP), jnp.int32),
    ],
    compiler_params=pltpu.CompilerParams(needs_layout_passes=False),
)
def _sc_aggregate(g_hbm, dst_hbm, zero_hbm, out_agg, out_cnt, out_win,
                  idx_v, rows_v, cnt_v, win_v, ma_v, mb_v, agg_sh, win_sh):
    cid = lax.axis_index("c")
    sid = lax.axis_index("s")
    wid = sid * NC + cid
    lane = lax.iota(jnp.int32, 16)
    ones = jnp.ones((16,), jnp.float32)
    lane_masks = [lane == l for l in range(16)]

    def _init(i, _):
        cnt_v[pl.ds(i * 16, 16)] = jnp.zeros((16,), jnp.float32)
        win_v[pl.ds(i * 16, 16)] = jnp.full((16,), -1, jnp.int32)
        return 0

    lax.fori_loop(0, NP // 16, _init, 0)

    pltpu.sync_copy(zero_hbm, agg_sh.at[pl.ds(sid * NSL, NSL)])
    plsc.subcore_barrier()

    def _chunk(i, _):
        base = wid * EC + i * CB
        pltpu.sync_copy(dst_hbm.at[pl.ds(base, CB)], idx_v)
        pltpu.sync_copy(g_hbm.at[pl.ds(base, CB)], rows_v)
        pltpu.sync_copy(rows_v, agg_sh.at[idx_v], add=True)
        for j in range(CB // 16):
            d16 = idx_v[pl.ds(j * 16, 16)]
            plsc.addupdate_scatter(cnt_v, [d16], ones)
            e16 = lane + (base + j * 16)
            for l in range(16):
                plsc.store_scatter(win_v, [d16], e16, mask=lane_masks[l])
        return 0

    lax.fori_loop(0, NCH, _chunk, 0)
    plsc.subcore_barrier()

    pltpu.sync_copy(win_v, win_sh.at[sid])
    plsc.subcore_barrier()
    nbase = sid * NSL
    pltpu.sync_copy(win_sh.at[0, pl.ds(nbase, NSL)], ma_v)
    for t in range(1, NS):
        pltpu.sync_copy(win_sh.at[t, pl.ds(nbase, NSL)], mb_v)

        def _mx(k, _, t=t):
            ma_v[pl.ds(k * 16, 16)] = jnp.maximum(ma_v[pl.ds(k * 16, 16)],
                                                  mb_v[pl.ds(k * 16, 16)])
            return 0

        lax.fori_loop(0, NSL // 16, _mx, 0)
    pltpu.sync_copy(ma_v, out_win.at[cid, pl.ds(nbase, NSL)])
    pltpu.sync_copy(cnt_v, out_cnt.at[wid])
    pltpu.sync_copy(agg_sh.at[pl.ds(nbase, NSL)], out_agg.at[cid, pl.ds(nbase, NSL)])


def kernel(node_features, edge_index, edge_attr, msg_w1, msg_b1, msg_w2, msg_b2,
           q_w, q_b, kv_w, kv_b, out_w1, out_b1, out_w2, out_b2, ln_g, ln_b):
    x = node_features
    src, dst = edge_index[0], edge_index[1]

    w_cat = jnp.concatenate([msg_w1[:DIN], msg_w1[DIN:2 * DIN], q_w,
                             kv_w[:DIN]], axis=1)
    P = _node_proj(x, w_cat)
    A, B, Q, K = (P[:, :DOUT], P[:, DOUT:2 * DOUT],
                  P[:, 2 * DOUT:3 * DOUT] + q_b, P[:, 3 * DOUT:])

    s1 = A[src] + B[dst]
    g = _edge_gelu(s1, edge_attr, msg_w1[2 * DIN:], msg_b1)

    zero_slab = jnp.zeros((NSL, DOUT), jnp.float32)
    agg2, cnt32, win2 = _sc_aggregate(g, dst, zero_slab)
    aggsum = agg2[0, :N] + agg2[1, :N]
    cnt = cnt32[:, :N].sum(axis=0)
    win = jnp.maximum(win2[0, :N], win2[1, :N])

    winc = jnp.maximum(win, 0)
    kvw = K[src[winc]] + edge_attr[winc] @ kv_w[DIN:] + kv_b
    attn = (Q.reshape(N, H, HD) * kvw.reshape(N, H, HD)).sum(-1) * (HD ** -0.5)
    attn = jnp.where(win[:, None] >= 0, attn, -jnp.inf)
    w = jax.nn.softmax(attn, axis=0)

    return _final(x, aggsum, cnt, w, msg_w2, msg_b2, out_w1, out_b1, out_w2,
                  out_b2, ln_g, ln_b)


# speedup vs baseline: 4.1830x; 1.9750x over previous
import functools
import jax
import jax.numpy as jnp
from jax import lax
from jax.experimental import pallas as pl
from jax.experimental.pallas import tpu as pltpu
from jax.experimental.pallas import tpu_sc as plsc

N = 10000
E = 320000
DIN = 128
DOUT = 128
H = 8
HD = DOUT // H
DE = 16

NC = 2
NS = 16
NW = NC * NS
NP = 10240
NSL = NP // NS
EC = E // NW
CB = 80
NCH = EC // CB


def _erf(x):
    p = 0.3275911
    a1, a2, a3, a4, a5 = (0.254829592, -0.284496736, 1.421413741,
                          -1.453152027, 1.061405429)
    ax = jnp.abs(x)
    t = 1.0 / (1.0 + p * ax)
    poly = ((((a5 * t + a4) * t + a3) * t + a2) * t + a1) * t
    y = 1.0 - poly * jnp.exp(-ax * ax)
    return jnp.sign(x) * y


def _gelu(x):
    return 0.5 * x * (1.0 + _erf(x * 0.7071067811865476))


def _proj_body(x_ref, w_ref, o_ref):
    o_ref[...] = jnp.dot(x_ref[...], w_ref[...],
                         preferred_element_type=jnp.float32)


def _node_proj(x, w_cat):
    bn = 2000
    return pl.pallas_call(
        _proj_body,
        grid=(N // bn,),
        in_specs=[pl.BlockSpec((bn, DIN), lambda i: (i, 0)),
                  pl.BlockSpec((DIN, 4 * DOUT), lambda i: (0, 0))],
        out_specs=pl.BlockSpec((bn, 4 * DOUT), lambda i: (i, 0)),
        out_shape=jax.ShapeDtypeStruct((N, 4 * DOUT), jnp.float32),
    )(x, w_cat)


def _edge_body(s1a_ref, s1b_ref, ea_ref, w1e_ref, b1_ref, o_ref):
    m1 = s1a_ref[...] + s1b_ref[...] + jnp.dot(
        ea_ref[...], w1e_ref[...],
        preferred_element_type=jnp.float32) + b1_ref[...]
    o_ref[...] = _gelu(m1)


def _edge_gelu(s1a, s1b, ea, w1e, b1):
    be = 4000
    return pl.pallas_call(
        _edge_body,
        grid=(E // be,),
        in_specs=[pl.BlockSpec((be, DOUT), lambda i: (i, 0)),
                  pl.BlockSpec((be, DOUT), lambda i: (i, 0)),
                  pl.BlockSpec((be, DE), lambda i: (i, 0)),
                  pl.BlockSpec((DE, DOUT), lambda i: (0, 0)),
                  pl.BlockSpec((1, DOUT), lambda i: (0, 0))],
        out_specs=pl.BlockSpec((be, DOUT), lambda i: (i, 0)),
        out_shape=jax.ShapeDtypeStruct((E, DOUT), jnp.float32),
    )(s1a, s1b, ea, w1e, b1.reshape(1, DOUT))


def _final_body(x_ref, aggs_ref, cnt_ref, w_ref, msgw2_ref, msgb2_ref,
                w1a_ref, w1b_ref, b1_ref, w2_ref, b2_ref, g_ref, bb_ref, o_ref):
    x = x_ref[...]
    agg = jnp.dot(aggs_ref[...], msgw2_ref[...],
                  preferred_element_type=jnp.float32)
    agg = agg + cnt_ref[...] * msgb2_ref[...]
    scale = jnp.repeat(w_ref[...], HD, axis=1)
    agg = agg * scale
    h = jnp.dot(x, w1a_ref[...], preferred_element_type=jnp.float32)
    h = h + jnp.dot(agg, w1b_ref[...], preferred_element_type=jnp.float32)
    h = _gelu(h + b1_ref[...])
    h = jnp.dot(h, w2_ref[...], preferred_element_type=jnp.float32) + b2_ref[...]
    h = x + h
    mu = jnp.mean(h, axis=-1, keepdims=True)
    var = jnp.mean((h - mu) ** 2, axis=-1, keepdims=True)
    o_ref[...] = (h - mu) * jax.lax.rsqrt(var + 1e-5) * g_ref[...] + bb_ref[...]


def _final(x, aggsum, cnt, w, msg_w2, msg_b2, out_w1, out_b1, out_w2, out_b2,
           ln_g, ln_b):
    bn = 2000
    row = lambda i: (i, 0)
    full = lambda i: (0, 0)
    return pl.pallas_call(
        _final_body,
        grid=(N // bn,),
        in_specs=[pl.BlockSpec((bn, DIN), row),
                  pl.BlockSpec((bn, DOUT), row),
                  pl.BlockSpec((bn, 1), row),
                  pl.BlockSpec((bn, H), row),
                  pl.BlockSpec((DOUT, DOUT), full),
                  pl.BlockSpec((1, DOUT), full),
                  pl.BlockSpec((DIN, DOUT), full),
                  pl.BlockSpec((DOUT, DOUT), full),
                  pl.BlockSpec((1, DOUT), full),
                  pl.BlockSpec((DOUT, DOUT), full),
                  pl.BlockSpec((1, DOUT), full),
                  pl.BlockSpec((1, DOUT), full),
                  pl.BlockSpec((1, DOUT), full)],
        out_specs=pl.BlockSpec((bn, DOUT), row),
        out_shape=jax.ShapeDtypeStruct((N, DOUT), jnp.float32),
    )(x, aggsum, cnt.reshape(N, 1), w, msg_w2, msg_b2.reshape(1, DOUT),
      out_w1[:DIN], out_w1[DIN:], out_b1.reshape(1, DOUT), out_w2,
      out_b2.reshape(1, DOUT), ln_g.reshape(1, DOUT), ln_b.reshape(1, DOUT))


_SC_MESH = plsc.VectorSubcoreMesh(core_axis_name="c", subcore_axis_name="s")


@functools.partial(
    pl.kernel,
    mesh=_SC_MESH,
    out_type=[
        jax.ShapeDtypeStruct((E, DOUT), jnp.float32),
        jax.ShapeDtypeStruct((E, DOUT), jnp.float32),
    ],
    scratch_types=[
        pltpu.VMEM((CB,), jnp.int32),
        pltpu.VMEM((CB,), jnp.int32),
        pltpu.VMEM((CB, DOUT), jnp.float32),
        pltpu.VMEM((CB, DOUT), jnp.float32),
        pltpu.SemaphoreType.DMA,
        pltpu.SemaphoreType.DMA,
    ],
    compiler_params=pltpu.CompilerParams(needs_layout_passes=False),
)
def _sc_gather(a_hbm, b_hbm, src_hbm, dst_hbm, out_a, out_b,
               idx_s, idx_d, rows_a, rows_b, sem_a, sem_b):
    cid = lax.axis_index("c")
    sid = lax.axis_index("s")
    wid = sid * NC + cid

    def _chunk(i, _):
        base = wid * EC + i * CB
        pltpu.sync_copy(src_hbm.at[pl.ds(base, CB)], idx_s)
        pltpu.sync_copy(dst_hbm.at[pl.ds(base, CB)], idx_d)
        ca = pltpu.async_copy(a_hbm.at[idx_s], rows_a, sem_a)
        cb = pltpu.async_copy(b_hbm.at[idx_d], rows_b, sem_b)
        ca.wait()
        cb.wait()
        pltpu.sync_copy(rows_a, out_a.at[pl.ds(base, CB)])
        pltpu.sync_copy(rows_b, out_b.at[pl.ds(base, CB)])
        return 0

    lax.fori_loop(0, NCH, _chunk, 0)


@functools.partial(
    pl.kernel,
    mesh=_SC_MESH,
    out_type=[
        jax.ShapeDtypeStruct((NC, NP, DOUT), jnp.float32),
        jax.ShapeDtypeStruct((NW, NP), jnp.float32),
        jax.ShapeDtypeStruct((NC, NP), jnp.int32),
    ],
    scratch_types=[
        pltpu.VMEM((CB,), jnp.int32),
        pltpu.VMEM((CB, DOUT), jnp.float32),
        pltpu.VMEM((NP,), jnp.float32),
        pltpu.VMEM((NP,), jnp.int32),
        pltpu.VMEM((NSL,), jnp.int32),
        pltpu.VMEM((NSL,), jnp.int32),
        pltpu.VMEM_SHARED((NP, DOUT), jnp.float32),
        pltpu.VMEM_SHARED((NS, NP), jnp.int32),
    ],
    compiler_params=pltpu.CompilerParams(needs_layout_passes=False),
)
def _sc_aggregate(g_hbm, dst_hbm, zero_hbm, out_agg, out_cnt, out_win,
                  idx_v, rows_v, cnt_v, win_v, ma_v, mb_v, agg_sh, win_sh):
    cid = lax.axis_index("c")
    sid = lax.axis_index("s")
    wid = sid * NC + cid
    lane = lax.iota(jnp.int32, 16)
    ones = jnp.ones((16,), jnp.float32)
    lane_masks = [lane == l for l in range(16)]

    def _init(i, _):
        cnt_v[pl.ds(i * 16, 16)] = jnp.zeros((16,), jnp.float32)
        win_v[pl.ds(i * 16, 16)] = jnp.full((16,), -1, jnp.int32)
        return 0

    lax.fori_loop(0, NP // 16, _init, 0)

    pltpu.sync_copy(zero_hbm, agg_sh.at[pl.ds(sid * NSL, NSL)])
    plsc.subcore_barrier()

    def _chunk(i, _):
        base = wid * EC + i * CB
        pltpu.sync_copy(dst_hbm.at[pl.ds(base, CB)], idx_v)
        pltpu.sync_copy(g_hbm.at[pl.ds(base, CB)], rows_v)
        pltpu.sync_copy(rows_v, agg_sh.at[idx_v], add=True)
        for j in range(CB // 16):
            d16 = idx_v[pl.ds(j * 16, 16)]
            plsc.addupdate_scatter(cnt_v, [d16], ones)
            e16 = lane + (base + j * 16)
            for l in range(16):
                plsc.store_scatter(win_v, [d16], e16, mask=lane_masks[l])
        return 0

    lax.fori_loop(0, NCH, _chunk, 0)
    plsc.subcore_barrier()

    pltpu.sync_copy(win_v, win_sh.at[sid])
    plsc.subcore_barrier()
    nbase = sid * NSL
    pltpu.sync_copy(win_sh.at[0, pl.ds(nbase, NSL)], ma_v)
    for t in range(1, NS):
        pltpu.sync_copy(win_sh.at[t, pl.ds(nbase, NSL)], mb_v)

        def _mx(k, _, t=t):
            ma_v[pl.ds(k * 16, 16)] = jnp.maximum(ma_v[pl.ds(k * 16, 16)],
                                                  mb_v[pl.ds(k * 16, 16)])
            return 0

        lax.fori_loop(0, NSL // 16, _mx, 0)
    pltpu.sync_copy(ma_v, out_win.at[cid, pl.ds(nbase, NSL)])
    pltpu.sync_copy(cnt_v, out_cnt.at[wid])
    pltpu.sync_copy(agg_sh.at[pl.ds(nbase, NSL)], out_agg.at[cid, pl.ds(nbase, NSL)])


def kernel(node_features, edge_index, edge_attr, msg_w1, msg_b1, msg_w2, msg_b2,
           q_w, q_b, kv_w, kv_b, out_w1, out_b1, out_w2, out_b2, ln_g, ln_b):
    x = node_features
    src, dst = edge_index[0], edge_index[1]

    w_cat = jnp.concatenate([msg_w1[:DIN], msg_w1[DIN:2 * DIN], q_w,
                             kv_w[:DIN]], axis=1)
    P = _node_proj(x, w_cat)
    A, B, Q, K = (P[:, :DOUT], P[:, DOUT:2 * DOUT],
                  P[:, 2 * DOUT:3 * DOUT] + q_b, P[:, 3 * DOUT:])

    s1a, s1b = _sc_gather(A, B, src, dst)
    g = _edge_gelu(s1a, s1b, edge_attr, msg_w1[2 * DIN:], msg_b1)

    zero_slab = jnp.zeros((NSL, DOUT), jnp.float32)
    agg2, cnt32, win2 = _sc_aggregate(g, dst, zero_slab)
    aggsum = agg2[0, :N] + agg2[1, :N]
    cnt = cnt32[:, :N].sum(axis=0)
    win = jnp.maximum(win2[0, :N], win2[1, :N])

    winc = jnp.maximum(win, 0)
    kvw = K[src[winc]] + edge_attr[winc] @ kv_w[DIN:] + kv_b
    attn = (Q.reshape(N, H, HD) * kvw.reshape(N, H, HD)).sum(-1) * (HD ** -0.5)
    attn = jnp.where(win[:, None] >= 0, attn, -jnp.inf)
    w = jax.nn.softmax(attn, axis=0)

    return _final(x, aggsum, cnt, w, msg_w2, msg_b2, out_w1, out_b1, out_w2,
                  out_b2, ln_g, ln_b)


# speedup vs baseline: 4.1994x; 1.0039x over previous
import functools
import jax
import jax.numpy as jnp
from jax import lax
from jax.experimental import pallas as pl
from jax.experimental.pallas import tpu as pltpu
from jax.experimental.pallas import tpu_sc as plsc

N = 10000
E = 320000
DIN = 128
DOUT = 128
H = 8
HD = DOUT // H
DE = 16

NC = 2
NS = 16
NW = NC * NS
NP = 10240
NSL = NP // NS
EC = E // NW
CB = 80
NCH = EC // CB


def _erf(x):
    p = 0.3275911
    a1, a2, a3, a4, a5 = (0.254829592, -0.284496736, 1.421413741,
                          -1.453152027, 1.061405429)
    ax = jnp.abs(x)
    t = 1.0 / (1.0 + p * ax)
    poly = ((((a5 * t + a4) * t + a3) * t + a2) * t + a1) * t
    y = 1.0 - poly * jnp.exp(-ax * ax)
    return jnp.sign(x) * y


def _gelu(x):
    return 0.5 * x * (1.0 + _erf(x * 0.7071067811865476))


def _proj_body(x_ref, w_ref, o_ref):
    o_ref[...] = jnp.dot(x_ref[...], w_ref[...],
                         preferred_element_type=jnp.float32)


def _node_proj(x, w_cat):
    bn = 2000
    return pl.pallas_call(
        _proj_body,
        grid=(N // bn,),
        in_specs=[pl.BlockSpec((bn, DIN), lambda i: (i, 0)),
                  pl.BlockSpec((DIN, 4 * DOUT), lambda i: (0, 0))],
        out_specs=pl.BlockSpec((bn, 4 * DOUT), lambda i: (i, 0)),
        out_shape=jax.ShapeDtypeStruct((N, 4 * DOUT), jnp.float32),
    )(x, w_cat)


def _edge_body(s1a_ref, s1b_ref, ea_ref, w1e_ref, b1_ref, o_ref):
    m1 = s1a_ref[...] + s1b_ref[...] + jnp.dot(
        ea_ref[...], w1e_ref[...],
        preferred_element_type=jnp.float32) + b1_ref[...]
    o_ref[...] = _gelu(m1)


def _edge_gelu(s1a, s1b, ea, w1e, b1):
    be = 4000
    return pl.pallas_call(
        _edge_body,
        grid=(E // be,),
        in_specs=[pl.BlockSpec((be, DOUT), lambda i: (i, 0)),
                  pl.BlockSpec((be, DOUT), lambda i: (i, 0)),
                  pl.BlockSpec((be, DE), lambda i: (i, 0)),
                  pl.BlockSpec((DE, DOUT), lambda i: (0, 0)),
                  pl.BlockSpec((1, DOUT), lambda i: (0, 0))],
        out_specs=pl.BlockSpec((be, DOUT), lambda i: (i, 0)),
        out_shape=jax.ShapeDtypeStruct((E, DOUT), jnp.float32),
    )(s1a, s1b, ea, w1e, b1.reshape(1, DOUT))


def _final_body(x_ref, agg0_ref, agg1_ref, cnt_ref, b2m_ref, w_ref, msgw2_ref,
                w1a_ref, w1b_ref, b1_ref, w2_ref, b2_ref, g_ref, bb_ref, o_ref):
    x = x_ref[...]
    aggs = agg0_ref[0] + agg1_ref[0]
    agg = jnp.dot(aggs, msgw2_ref[...], preferred_element_type=jnp.float32)
    agg = agg + jnp.dot(cnt_ref[...], b2m_ref[...],
                        preferred_element_type=jnp.float32)
    scale = jnp.repeat(w_ref[...], HD, axis=1)
    agg = agg * scale
    h = jnp.dot(x, w1a_ref[...], preferred_element_type=jnp.float32)
    h = h + jnp.dot(agg, w1b_ref[...], preferred_element_type=jnp.float32)
    h = _gelu(h + b1_ref[...])
    h = jnp.dot(h, w2_ref[...], preferred_element_type=jnp.float32) + b2_ref[...]
    h = x + h
    mu = jnp.mean(h, axis=-1, keepdims=True)
    var = jnp.mean((h - mu) ** 2, axis=-1, keepdims=True)
    o_ref[...] = (h - mu) * jax.lax.rsqrt(var + 1e-5) * g_ref[...] + bb_ref[...]


def _final(x, agg2, cnt32, w, msg_w2, msg_b2, out_w1, out_b1, out_w2, out_b2,
           ln_g, ln_b):
    bn = 2000
    row = lambda i: (i, 0)
    full = lambda i: (0, 0)
    b2m = jnp.broadcast_to(msg_b2, (NW, DOUT))
    return pl.pallas_call(
        _final_body,
        grid=(N // bn,),
        in_specs=[pl.BlockSpec((bn, DIN), row),
                  pl.BlockSpec((1, bn, DOUT), lambda i: (0, i, 0)),
                  pl.BlockSpec((1, bn, DOUT), lambda i: (1, i, 0)),
                  pl.BlockSpec((bn, NW), lambda i: (i, 0)),
                  pl.BlockSpec((NW, DOUT), full),
                  pl.BlockSpec((bn, H), row),
                  pl.BlockSpec((DOUT, DOUT), full),
                  pl.BlockSpec((DIN, DOUT), full),
                  pl.BlockSpec((DOUT, DOUT), full),
                  pl.BlockSpec((1, DOUT), full),
                  pl.BlockSpec((DOUT, DOUT), full),
                  pl.BlockSpec((1, DOUT), full),
                  pl.BlockSpec((1, DOUT), full),
                  pl.BlockSpec((1, DOUT), full)],
        out_specs=pl.BlockSpec((bn, DOUT), row),
        out_shape=jax.ShapeDtypeStruct((N, DOUT), jnp.float32),
    )(x, agg2, agg2, cnt32.T, b2m, w, msg_w2,
      out_w1[:DIN], out_w1[DIN:], out_b1.reshape(1, DOUT), out_w2,
      out_b2.reshape(1, DOUT), ln_g.reshape(1, DOUT), ln_b.reshape(1, DOUT))


_SC_MESH = plsc.VectorSubcoreMesh(core_axis_name="c", subcore_axis_name="s")


@functools.partial(
    pl.kernel,
    mesh=_SC_MESH,
    out_type=[
        jax.ShapeDtypeStruct((E, DOUT), jnp.float32),
        jax.ShapeDtypeStruct((E, DOUT), jnp.float32),
    ],
    scratch_types=[
        pltpu.VMEM((CB,), jnp.int32),
        pltpu.VMEM((CB,), jnp.int32),
        pltpu.VMEM((CB, DOUT), jnp.float32),
        pltpu.VMEM((CB, DOUT), jnp.float32),
        pltpu.SemaphoreType.DMA,
        pltpu.SemaphoreType.DMA,
    ],
    compiler_params=pltpu.CompilerParams(needs_layout_passes=False),
)
def _sc_gather(a_hbm, b_hbm, src_hbm, dst_hbm, out_a, out_b,
               idx_s, idx_d, rows_a, rows_b, sem_a, sem_b):
    cid = lax.axis_index("c")
    sid = lax.axis_index("s")
    wid = sid * NC + cid

    def _chunk(i, _):
        base = wid * EC + i * CB
        pltpu.sync_copy(src_hbm.at[pl.ds(base, CB)], idx_s)
        pltpu.sync_copy(dst_hbm.at[pl.ds(base, CB)], idx_d)
        ca = pltpu.async_copy(a_hbm.at[idx_s], rows_a, sem_a)
        cb = pltpu.async_copy(b_hbm.at[idx_d], rows_b, sem_b)
        ca.wait()
        cb.wait()
        pltpu.sync_copy(rows_a, out_a.at[pl.ds(base, CB)])
        pltpu.sync_copy(rows_b, out_b.at[pl.ds(base, CB)])
        return 0

    lax.fori_loop(0, NCH, _chunk, 0)


@functools.partial(
    pl.kernel,
    mesh=_SC_MESH,
    out_type=[
        jax.ShapeDtypeStruct((NC, NP, DOUT), jnp.float32),
        jax.ShapeDtypeStruct((NW, NP), jnp.float32),
        jax.ShapeDtypeStruct((NC * NP,), jnp.int32),
    ],
    scratch_types=[
        pltpu.VMEM((CB,), jnp.int32),
        pltpu.VMEM((CB, DOUT), jnp.float32),
        pltpu.VMEM((NP,), jnp.float32),
        pltpu.VMEM((NP,), jnp.int32),
        pltpu.VMEM((NSL,), jnp.int32),
        pltpu.VMEM((NSL,), jnp.int32),
        pltpu.VMEM_SHARED((NP, DOUT), jnp.float32),
        pltpu.VMEM_SHARED((NS, NP), jnp.int32),
    ],
    compiler_params=pltpu.CompilerParams(needs_layout_passes=False),
)
def _sc_aggregate(g_hbm, dst_hbm, zero_hbm, out_agg, out_cnt, out_win,
                  idx_v, rows_v, cnt_v, win_v, ma_v, mb_v, agg_sh, win_sh):
    cid = lax.axis_index("c")
    sid = lax.axis_index("s")
    wid = sid * NC + cid
    lane = lax.iota(jnp.int32, 16)
    ones = jnp.ones((16,), jnp.float32)
    lane_masks = [lane == l for l in range(16)]

    def _init(i, _):
        cnt_v[pl.ds(i * 16, 16)] = jnp.zeros((16,), jnp.float32)
        win_v[pl.ds(i * 16, 16)] = jnp.full((16,), -1, jnp.int32)
        return 0

    lax.fori_loop(0, NP // 16, _init, 0)

    pltpu.sync_copy(zero_hbm, agg_sh.at[pl.ds(sid * NSL, NSL)])
    plsc.subcore_barrier()

    def _chunk(i, _):
        base = wid * EC + i * CB
        pltpu.sync_copy(dst_hbm.at[pl.ds(base, CB)], idx_v)
        pltpu.sync_copy(g_hbm.at[pl.ds(base, CB)], rows_v)
        pltpu.sync_copy(rows_v, agg_sh.at[idx_v], add=True)
        for j in range(CB // 16):
            d16 = idx_v[pl.ds(j * 16, 16)]
            plsc.addupdate_scatter(cnt_v, [d16], ones)
            e16 = lane + (base + j * 16)
            for l in range(16):
                plsc.store_scatter(win_v, [d16], e16, mask=lane_masks[l])
        return 0

    lax.fori_loop(0, NCH, _chunk, 0)
    plsc.subcore_barrier()

    pltpu.sync_copy(win_v, win_sh.at[sid])
    plsc.subcore_barrier()
    nbase = sid * NSL
    pltpu.sync_copy(win_sh.at[0, pl.ds(nbase, NSL)], ma_v)
    for t in range(1, NS):
        pltpu.sync_copy(win_sh.at[t, pl.ds(nbase, NSL)], mb_v)

        def _mx(k, _, t=t):
            ma_v[pl.ds(k * 16, 16)] = jnp.maximum(ma_v[pl.ds(k * 16, 16)],
                                                  mb_v[pl.ds(k * 16, 16)])
            return 0

        lax.fori_loop(0, NSL // 16, _mx, 0)
    pltpu.sync_copy(ma_v, out_win.at[pl.ds(cid * NP + nbase, NSL)])
    pltpu.sync_copy(cnt_v, out_cnt.at[wid])
    pltpu.sync_copy(agg_sh.at[pl.ds(nbase, NSL)], out_agg.at[cid, pl.ds(nbase, NSL)])


NPW = NP // NW
WGB = 80


E8 = E // 8


@functools.partial(
    pl.kernel,
    mesh=_SC_MESH,
    out_type=[
        jax.ShapeDtypeStruct((NP, DOUT), jnp.float32),
        jax.ShapeDtypeStruct((NP, 128), jnp.float32),
        jax.ShapeDtypeStruct((NP,), jnp.int32),
    ],
    scratch_types=[
        pltpu.VMEM((NPW,), jnp.int32),
        pltpu.VMEM((NPW,), jnp.int32),
        pltpu.VMEM((NPW,), jnp.int32),
        pltpu.VMEM((NPW,), jnp.int32),
        pltpu.VMEM((WGB,), jnp.int32),
        pltpu.VMEM((WGB, 128), jnp.int32),
        pltpu.VMEM((WGB, DOUT), jnp.float32),
        pltpu.VMEM((WGB, 128), jnp.float32),
        pltpu.SemaphoreType.DMA,
    ],
    compiler_params=pltpu.CompilerParams(needs_layout_passes=False),
)
def _sc_winner_gather(win2_hbm, srcp_hbm, ea8_hbm, k_hbm,
                      out_k, out_ea, out_win,
                      wa_v, wb_v, wc_v, wd_v, sw_v, srcrows_v, krows_v,
                      earows_v, sem):
    cid = lax.axis_index("c")
    sid = lax.axis_index("s")
    wid = sid * NC + cid
    nbase = wid * NPW
    lane = lax.iota(jnp.int32, 16)
    pltpu.sync_copy(win2_hbm.at[pl.ds(nbase, NPW)], wa_v)
    pltpu.sync_copy(win2_hbm.at[pl.ds(NP + nbase, NPW)], wb_v)

    def _mx(k, _):
        m = jnp.maximum(wa_v[pl.ds(k * 16, 16)], wb_v[pl.ds(k * 16, 16)])
        wa_v[pl.ds(k * 16, 16)] = m
        c = jnp.maximum(m, 0)
        wc_v[pl.ds(k * 16, 16)] = c
        wd_v[pl.ds(k * 16, 16)] = c >> 3
        return 0

    lax.fori_loop(0, NPW // 16, _mx, 0)
    pltpu.sync_copy(wa_v, out_win.at[pl.ds(nbase, NPW)])
    for k in range(NPW // WGB):
        idx8 = wd_v.at[pl.ds(k * WGB, WGB)]
        pltpu.async_copy(srcp_hbm.at[idx8], srcrows_v, sem).wait()
        pltpu.async_copy(ea8_hbm.at[idx8], earows_v, sem).wait()
        for t in range(WGB // 16):
            cols = wc_v[pl.ds(k * WGB + t * 16, 16)] & 7
            rows = lane + (t * 16)
            sw_v[pl.ds(t * 16, 16)] = plsc.load_gather(srcrows_v, [rows, cols])
        pltpu.async_copy(k_hbm.at[sw_v], krows_v, sem).wait()
        pltpu.sync_copy(earows_v, out_ea.at[pl.ds(nbase + k * WGB, WGB)])
        pltpu.sync_copy(krows_v, out_k.at[pl.ds(nbase + k * WGB, WGB)])


def _attn_body(q_ref, qb_ref, kw_ref, ea_ref, win_ref, kvw_ref, kvb_ref,
               seg_ref, o_ref):
    q = q_ref[...] + qb_ref[...]
    sub = jax.lax.broadcasted_iota(jnp.int32, (N, 128), 1) // DE
    ea = jnp.where(sub == (win_ref[...] & 7), ea_ref[...], 0.0)
    kv = kw_ref[...] + jnp.dot(ea, kvw_ref[...],
                               preferred_element_type=jnp.float32) + kvb_ref[...]
    prod = q * kv
    attn = jnp.dot(prod, seg_ref[...], preferred_element_type=jnp.float32)
    attn = attn * (HD ** -0.5)
    valid = win_ref[...] >= 0
    attn = jnp.where(valid, attn, -jnp.inf)
    mx = jnp.max(attn, axis=0, keepdims=True)
    ex = jnp.where(valid, jnp.exp(attn - mx), 0.0)
    o_ref[...] = ex / jnp.sum(ex, axis=0, keepdims=True)


def _attention(q_raw, q_b, kw, eaw, winm, kv_we, kv_b, seg):
    full = lambda: None
    return pl.pallas_call(
        _attn_body,
        grid=(1,),
        in_specs=[pl.BlockSpec((N, DOUT), lambda i: (0, 0)),
                  pl.BlockSpec((1, DOUT), lambda i: (0, 0)),
                  pl.BlockSpec((N, DOUT), lambda i: (0, 0)),
                  pl.BlockSpec((N, 128), lambda i: (0, 0)),
                  pl.BlockSpec((N, 1), lambda i: (0, 0)),
                  pl.BlockSpec((128, DOUT), lambda i: (0, 0)),
                  pl.BlockSpec((1, DOUT), lambda i: (0, 0)),
                  pl.BlockSpec((DOUT, H), lambda i: (0, 0))],
        out_specs=pl.BlockSpec((N, H), lambda i: (0, 0)),
        out_shape=jax.ShapeDtypeStruct((N, H), jnp.float32),
    )(q_raw, q_b.reshape(1, DOUT), kw, eaw, winm.reshape(N, 1), kv_we,
      kv_b.reshape(1, DOUT), seg)


def kernel(node_features, edge_index, edge_attr, msg_w1, msg_b1, msg_w2, msg_b2,
           q_w, q_b, kv_w, kv_b, out_w1, out_b1, out_w2, out_b2, ln_g, ln_b):
    x = node_features
    src, dst = edge_index[0], edge_index[1]

    w_cat = jnp.concatenate([msg_w1[:DIN], msg_w1[DIN:2 * DIN], q_w,
                             kv_w[:DIN]], axis=1)
    P = _node_proj(x, w_cat)
    A, B, Qr, K = (P[:, :DOUT], P[:, DOUT:2 * DOUT],
                   P[:, 2 * DOUT:3 * DOUT], P[:, 3 * DOUT:])

    s1a, s1b = _sc_gather(A, B, src, dst)
    g = _edge_gelu(s1a, s1b, edge_attr, msg_w1[2 * DIN:], msg_b1)

    zero_slab = jnp.zeros((NSL, DOUT), jnp.float32)
    agg2, cnt32, win2 = _sc_aggregate(g, dst, zero_slab)

    srcp = jnp.pad(src.reshape(E8, 8), ((0, 0), (0, 120)))
    ea8 = edge_attr.reshape(E8, 128)
    kw, eaw8, winm = _sc_winner_gather(win2, srcp, ea8, K)
    seg = jnp.repeat(jnp.eye(H, dtype=jnp.float32), HD, axis=0)
    kv128 = jnp.tile(kv_w[DIN:], (8, 1))
    w = _attention(Qr, q_b, kw[:N], eaw8[:N], winm[:N], kv128, kv_b, seg)

    return _final(x, agg2, cnt32, w, msg_w2, msg_b2, out_w1, out_b1, out_w2,
                  out_b2, ln_g, ln_b)


# speedup vs baseline: 5.2463x; 1.2493x over previous
import functools
import jax
import jax.numpy as jnp
from jax import lax
from jax.experimental import pallas as pl
from jax.experimental.pallas import tpu as pltpu
from jax.experimental.pallas import tpu_sc as plsc

N = 10000
E = 320000
DIN = 128
DOUT = 128
H = 8
HD = DOUT // H
DE = 16

NC = 2
NS = 16
NW = NC * NS
NP = 10240
NSL = NP // NS
EC = E // NW
CB = 80
NCH = EC // CB


def _erf(x):
    p = 0.3275911
    a1, a2, a3, a4, a5 = (0.254829592, -0.284496736, 1.421413741,
                          -1.453152027, 1.061405429)
    ax = jnp.abs(x)
    t = 1.0 / (1.0 + p * ax)
    poly = ((((a5 * t + a4) * t + a3) * t + a2) * t + a1) * t
    y = 1.0 - poly * jnp.exp(-ax * ax)
    return jnp.sign(x) * y


def _gelu(x):
    return 0.5 * x * (1.0 + _erf(x * 0.7071067811865476))


def _proj_body(x_ref, w_ref, o_ref):
    o_ref[...] = jnp.dot(x_ref[...], w_ref[...],
                         preferred_element_type=jnp.float32)


def _node_proj(x, w_cat):
    bn = 2000
    return pl.pallas_call(
        _proj_body,
        grid=(N // bn,),
        in_specs=[pl.BlockSpec((bn, DIN), lambda i: (i, 0)),
                  pl.BlockSpec((DIN, 4 * DOUT), lambda i: (0, 0))],
        out_specs=pl.BlockSpec((bn, 4 * DOUT), lambda i: (i, 0)),
        out_shape=jax.ShapeDtypeStruct((N, 4 * DOUT), jnp.float32),
    )(x, w_cat)


def _edge_body(s1_ref, ea_ref, w1e_ref, b1_ref, o_ref):
    m1 = s1_ref[...] + jnp.dot(
        ea_ref[...], w1e_ref[...],
        preferred_element_type=jnp.float32) + b1_ref[...]
    o_ref[...] = _gelu(m1)


def _edge_gelu(s1, ea, w1e, b1):
    be = 4000
    return pl.pallas_call(
        _edge_body,
        grid=(E // be,),
        in_specs=[pl.BlockSpec((be, DOUT), lambda i: (i, 0)),
                  pl.BlockSpec((be, DE), lambda i: (i, 0)),
                  pl.BlockSpec((DE, DOUT), lambda i: (0, 0)),
                  pl.BlockSpec((1, DOUT), lambda i: (0, 0))],
        out_specs=pl.BlockSpec((be, DOUT), lambda i: (i, 0)),
        out_shape=jax.ShapeDtypeStruct((E, DOUT), jnp.float32),
    )(s1, ea, w1e, b1.reshape(1, DOUT))


def _final_body(x_ref, agg0_ref, agg1_ref, cnt_ref, b2m_ref, w_ref, msgw2_ref,
                w1a_ref, w1b_ref, b1_ref, w2_ref, b2_ref, g_ref, bb_ref, o_ref):
    x = x_ref[...]
    aggs = agg0_ref[0] + agg1_ref[0]
    agg = jnp.dot(aggs, msgw2_ref[...], preferred_element_type=jnp.float32)
    agg = agg + jnp.dot(cnt_ref[...], b2m_ref[...],
                        preferred_element_type=jnp.float32)
    scale = jnp.repeat(w_ref[...], HD, axis=1)
    agg = agg * scale
    h = jnp.dot(x, w1a_ref[...], preferred_element_type=jnp.float32)
    h = h + jnp.dot(agg, w1b_ref[...], preferred_element_type=jnp.float32)
    h = _gelu(h + b1_ref[...])
    h = jnp.dot(h, w2_ref[...], preferred_element_type=jnp.float32) + b2_ref[...]
    h = x + h
    mu = jnp.mean(h, axis=-1, keepdims=True)
    var = jnp.mean((h - mu) ** 2, axis=-1, keepdims=True)
    o_ref[...] = (h - mu) * jax.lax.rsqrt(var + 1e-5) * g_ref[...] + bb_ref[...]


def _final(x, agg2, cnt32, w, msg_w2, msg_b2, out_w1, out_b1, out_w2, out_b2,
           ln_g, ln_b):
    bn = 2000
    row = lambda i: (i, 0)
    full = lambda i: (0, 0)
    b2m = jnp.broadcast_to(msg_b2, (NW, DOUT))
    return pl.pallas_call(
        _final_body,
        grid=(N // bn,),
        in_specs=[pl.BlockSpec((bn, DIN), row),
                  pl.BlockSpec((1, bn, DOUT), lambda i: (0, i, 0)),
                  pl.BlockSpec((1, bn, DOUT), lambda i: (1, i, 0)),
                  pl.BlockSpec((bn, NW), lambda i: (i, 0)),
                  pl.BlockSpec((NW, DOUT), full),
                  pl.BlockSpec((bn, H), row),
                  pl.BlockSpec((DOUT, DOUT), full),
                  pl.BlockSpec((DIN, DOUT), full),
                  pl.BlockSpec((DOUT, DOUT), full),
                  pl.BlockSpec((1, DOUT), full),
                  pl.BlockSpec((DOUT, DOUT), full),
                  pl.BlockSpec((1, DOUT), full),
                  pl.BlockSpec((1, DOUT), full),
                  pl.BlockSpec((1, DOUT), full)],
        out_specs=pl.BlockSpec((bn, DOUT), row),
        out_shape=jax.ShapeDtypeStruct((N, DOUT), jnp.float32),
    )(x, agg2, agg2, cnt32.T, b2m, w, msg_w2,
      out_w1[:DIN], out_w1[DIN:], out_b1.reshape(1, DOUT), out_w2,
      out_b2.reshape(1, DOUT), ln_g.reshape(1, DOUT), ln_b.reshape(1, DOUT))


_SC_MESH = plsc.VectorSubcoreMesh(core_axis_name="c", subcore_axis_name="s")


@functools.partial(
    pl.kernel,
    mesh=_SC_MESH,
    out_type=jax.ShapeDtypeStruct((E, DOUT), jnp.float32),
    scratch_types=[
        pltpu.VMEM((EC,), jnp.int32),
        pltpu.VMEM((EC,), jnp.int32),
        pltpu.VMEM((2, CB, DOUT), jnp.float32),
        pltpu.VMEM((2, CB, DOUT), jnp.float32),
        pltpu.SemaphoreType.DMA,
        pltpu.SemaphoreType.DMA,
        pltpu.SemaphoreType.DMA,
        pltpu.SemaphoreType.DMA,
    ],
    compiler_params=pltpu.CompilerParams(needs_layout_passes=False),
)
def _sc_gather(a_hbm, b_hbm, src_hbm, dst_hbm, out_s1,
               idxs_v, idxd_v, rows_a, rows_b, sem_a, sem_b, sem_w0, sem_w1):
    cid = lax.axis_index("c")
    sid = lax.axis_index("s")
    wid = sid * NC + cid
    ebase = wid * EC
    pltpu.sync_copy(src_hbm.at[pl.ds(ebase, EC)], idxs_v)
    pltpu.sync_copy(dst_hbm.at[pl.ds(ebase, EC)], idxd_v)
    sem_w = (sem_w0, sem_w1)

    def _start(i, s):
        pltpu.async_copy(a_hbm.at[idxs_v.at[pl.ds(i * CB, CB)]],
                         rows_a.at[s], sem_a)
        pltpu.async_copy(b_hbm.at[idxd_v.at[pl.ds(i * CB, CB)]],
                         rows_b.at[s], sem_b)

    def _step(i, s, prefetch):
        pltpu.make_async_copy(a_hbm.at[pl.ds(0, CB)], rows_a.at[s],
                              sem_a).wait()
        pltpu.make_async_copy(b_hbm.at[pl.ds(0, CB)], rows_b.at[s],
                              sem_b).wait()

        def _add(r, _):
            for c in range(DOUT // 16):
                sl = pl.ds(c * 16, 16)
                rows_a[s, r, sl] = rows_a[s, r, sl] + rows_b[s, r, sl]
            return 0

        lax.fori_loop(0, CB, _add, 0)
        dst_slice = out_s1.at[pl.ds(ebase + i * CB, CB)]
        pltpu.async_copy(rows_a.at[s], dst_slice, sem_w[s])
        if prefetch:
            @pl.when(i + 2 < NCH)
            def _():
                pltpu.make_async_copy(rows_a.at[s], dst_slice,
                                      sem_w[s]).wait()
                _start(i + 2, s)

    _start(0, 0)
    _start(1, 1)

    def _pair(i2, _):
        _step(2 * i2, 0, True)
        _step(2 * i2 + 1, 1, True)
        return 0

    lax.fori_loop(0, NCH // 2, _pair, 0)
    _step(NCH - 1, 0, False)
    pltpu.make_async_copy(rows_a.at[0], out_s1.at[pl.ds(0, CB)], sem_w0).wait()
    pltpu.make_async_copy(rows_a.at[1], out_s1.at[pl.ds(0, CB)], sem_w1).wait()


@functools.partial(
    pl.kernel,
    mesh=_SC_MESH,
    out_type=[
        jax.ShapeDtypeStruct((NC, NP, DOUT), jnp.float32),
        jax.ShapeDtypeStruct((NW, NP), jnp.float32),
        jax.ShapeDtypeStruct((NC * NP,), jnp.int32),
    ],
    scratch_types=[
        pltpu.VMEM((CB,), jnp.int32),
        pltpu.VMEM((CB, DOUT), jnp.float32),
        pltpu.VMEM((NP,), jnp.float32),
        pltpu.VMEM((NP,), jnp.int32),
        pltpu.VMEM((NSL,), jnp.int32),
        pltpu.VMEM((NSL,), jnp.int32),
        pltpu.VMEM_SHARED((NP, DOUT), jnp.float32),
        pltpu.VMEM_SHARED((NS, NP), jnp.int32),
    ],
    compiler_params=pltpu.CompilerParams(needs_layout_passes=False),
)
def _sc_aggregate(g_hbm, dst_hbm, zero_hbm, out_agg, out_cnt, out_win,
                  idx_v, rows_v, cnt_v, win_v, ma_v, mb_v, agg_sh, win_sh):
    cid = lax.axis_index("c")
    sid = lax.axis_index("s")
    wid = sid * NC + cid
    lane = lax.iota(jnp.int32, 16)
    ones = jnp.ones((16,), jnp.float32)
    lane_masks = [lane == l for l in range(16)]

    def _init(i, _):
        cnt_v[pl.ds(i * 16, 16)] = jnp.zeros((16,), jnp.float32)
        win_v[pl.ds(i * 16, 16)] = jnp.full((16,), -1, jnp.int32)
        return 0

    lax.fori_loop(0, NP // 16, _init, 0)

    pltpu.sync_copy(zero_hbm, agg_sh.at[pl.ds(sid * NSL, NSL)])
    plsc.subcore_barrier()

    def _chunk(i, _):
        base = wid * EC + i * CB
        pltpu.sync_copy(dst_hbm.at[pl.ds(base, CB)], idx_v)
        pltpu.sync_copy(g_hbm.at[pl.ds(base, CB)], rows_v)
        pltpu.sync_copy(rows_v, agg_sh.at[idx_v], add=True)
        for j in range(CB // 16):
            d16 = idx_v[pl.ds(j * 16, 16)]
            plsc.addupdate_scatter(cnt_v, [d16], ones)
            e16 = lane + (base + j * 16)
            for l in range(16):
                plsc.store_scatter(win_v, [d16], e16, mask=lane_masks[l])
        return 0

    lax.fori_loop(0, NCH, _chunk, 0)
    plsc.subcore_barrier()

    pltpu.sync_copy(win_v, win_sh.at[sid])
    plsc.subcore_barrier()
    nbase = sid * NSL
    pltpu.sync_copy(win_sh.at[0, pl.ds(nbase, NSL)], ma_v)
    for t in range(1, NS):
        pltpu.sync_copy(win_sh.at[t, pl.ds(nbase, NSL)], mb_v)

        def _mx(k, _, t=t):
            ma_v[pl.ds(k * 16, 16)] = jnp.maximum(ma_v[pl.ds(k * 16, 16)],
                                                  mb_v[pl.ds(k * 16, 16)])
            return 0

        lax.fori_loop(0, NSL // 16, _mx, 0)
    pltpu.sync_copy(ma_v, out_win.at[pl.ds(cid * NP + nbase, NSL)])
    pltpu.sync_copy(cnt_v, out_cnt.at[wid])
    pltpu.sync_copy(agg_sh.at[pl.ds(nbase, NSL)], out_agg.at[cid, pl.ds(nbase, NSL)])


NPW = NP // NW
WGB = 80


E8 = E // 8


@functools.partial(
    pl.kernel,
    mesh=_SC_MESH,
    out_type=[
        jax.ShapeDtypeStruct((NP, DOUT), jnp.float32),
        jax.ShapeDtypeStruct((NP, 128), jnp.float32),
        jax.ShapeDtypeStruct((NP,), jnp.int32),
    ],
    scratch_types=[
        pltpu.VMEM((NPW,), jnp.int32),
        pltpu.VMEM((NPW,), jnp.int32),
        pltpu.VMEM((NPW,), jnp.int32),
        pltpu.VMEM((NPW,), jnp.int32),
        pltpu.VMEM((WGB,), jnp.int32),
        pltpu.VMEM((WGB, 128), jnp.int32),
        pltpu.VMEM((WGB, DOUT), jnp.float32),
        pltpu.VMEM((WGB, 128), jnp.float32),
        pltpu.SemaphoreType.DMA,
    ],
    compiler_params=pltpu.CompilerParams(needs_layout_passes=False),
)
def _sc_winner_gather(win2_hbm, srcp_hbm, ea8_hbm, k_hbm,
                      out_k, out_ea, out_win,
                      wa_v, wb_v, wc_v, wd_v, sw_v, srcrows_v, krows_v,
                      earows_v, sem):
    cid = lax.axis_index("c")
    sid = lax.axis_index("s")
    wid = sid * NC + cid
    nbase = wid * NPW
    lane = lax.iota(jnp.int32, 16)
    pltpu.sync_copy(win2_hbm.at[pl.ds(nbase, NPW)], wa_v)
    pltpu.sync_copy(win2_hbm.at[pl.ds(NP + nbase, NPW)], wb_v)

    def _mx(k, _):
        m = jnp.maximum(wa_v[pl.ds(k * 16, 16)], wb_v[pl.ds(k * 16, 16)])
        wa_v[pl.ds(k * 16, 16)] = m
        c = jnp.maximum(m, 0)
        wc_v[pl.ds(k * 16, 16)] = c
        wd_v[pl.ds(k * 16, 16)] = c >> 3
        return 0

    lax.fori_loop(0, NPW // 16, _mx, 0)
    pltpu.sync_copy(wa_v, out_win.at[pl.ds(nbase, NPW)])
    for k in range(NPW // WGB):
        idx8 = wd_v.at[pl.ds(k * WGB, WGB)]
        pltpu.async_copy(srcp_hbm.at[idx8], srcrows_v, sem).wait()
        pltpu.async_copy(ea8_hbm.at[idx8], earows_v, sem).wait()
        for t in range(WGB // 16):
            cols = wc_v[pl.ds(k * WGB + t * 16, 16)] & 7
            rows = lane + (t * 16)
            sw_v[pl.ds(t * 16, 16)] = plsc.load_gather(srcrows_v, [rows, cols])
        pltpu.async_copy(k_hbm.at[sw_v], krows_v, sem).wait()
        pltpu.sync_copy(earows_v, out_ea.at[pl.ds(nbase + k * WGB, WGB)])
        pltpu.sync_copy(krows_v, out_k.at[pl.ds(nbase + k * WGB, WGB)])


def _attn_body(q_ref, qb_ref, kw_ref, ea_ref, win_ref, kvw_ref, kvb_ref,
               seg_ref, o_ref):
    q = q_ref[...] + qb_ref[...]
    sub = jax.lax.broadcasted_iota(jnp.int32, (N, 128), 1) // DE
    ea = jnp.where(sub == (win_ref[...] & 7), ea_ref[...], 0.0)
    kv = kw_ref[...] + jnp.dot(ea, kvw_ref[...],
                               preferred_element_type=jnp.float32) + kvb_ref[...]
    prod = q * kv
    attn = jnp.dot(prod, seg_ref[...], preferred_element_type=jnp.float32)
    attn = attn * (HD ** -0.5)
    valid = win_ref[...] >= 0
    attn = jnp.where(valid, attn, -jnp.inf)
    mx = jnp.max(attn, axis=0, keepdims=True)
    ex = jnp.where(valid, jnp.exp(attn - mx), 0.0)
    o_ref[...] = ex / jnp.sum(ex, axis=0, keepdims=True)


def _attention(q_raw, q_b, kw, eaw, winm, kv_we, kv_b, seg):
    full = lambda: None
    return pl.pallas_call(
        _attn_body,
        grid=(1,),
        in_specs=[pl.BlockSpec((N, DOUT), lambda i: (0, 0)),
                  pl.BlockSpec((1, DOUT), lambda i: (0, 0)),
                  pl.BlockSpec((N, DOUT), lambda i: (0, 0)),
                  pl.BlockSpec((N, 128), lambda i: (0, 0)),
                  pl.BlockSpec((N, 1), lambda i: (0, 0)),
                  pl.BlockSpec((128, DOUT), lambda i: (0, 0)),
                  pl.BlockSpec((1, DOUT), lambda i: (0, 0)),
                  pl.BlockSpec((DOUT, H), lambda i: (0, 0))],
        out_specs=pl.BlockSpec((N, H), lambda i: (0, 0)),
        out_shape=jax.ShapeDtypeStruct((N, H), jnp.float32),
    )(q_raw, q_b.reshape(1, DOUT), kw, eaw, winm.reshape(N, 1), kv_we,
      kv_b.reshape(1, DOUT), seg)


def kernel(node_features, edge_index, edge_attr, msg_w1, msg_b1, msg_w2, msg_b2,
           q_w, q_b, kv_w, kv_b, out_w1, out_b1, out_w2, out_b2, ln_g, ln_b):
    x = node_features
    src, dst = edge_index[0], edge_index[1]

    w_cat = jnp.concatenate([msg_w1[:DIN], msg_w1[DIN:2 * DIN], q_w,
                             kv_w[:DIN]], axis=1)
    P = _node_proj(x, w_cat)
    A, B, Qr, K = (P[:, :DOUT], P[:, DOUT:2 * DOUT],
                   P[:, 2 * DOUT:3 * DOUT], P[:, 3 * DOUT:])

    s1 = _sc_gather(A, B, src, dst)
    g = _edge_gelu(s1, edge_attr, msg_w1[2 * DIN:], msg_b1)

    zero_slab = jnp.zeros((NSL, DOUT), jnp.float32)
    agg2, cnt32, win2 = _sc_aggregate(g, dst, zero_slab)

    srcp = jnp.pad(src.reshape(E8, 8), ((0, 0), (0, 120)))
    ea8 = edge_attr.reshape(E8, 128)
    kw, eaw8, winm = _sc_winner_gather(win2, srcp, ea8, K)
    seg = jnp.repeat(jnp.eye(H, dtype=jnp.float32), HD, axis=0)
    kv128 = jnp.tile(kv_w[DIN:], (8, 1))
    w = _attention(Qr, q_b, kw[:N], eaw8[:N], winm[:N], kv128, kv_b, seg)

    return _final(x, agg2, cnt32, w, msg_w2, msg_b2, out_w1, out_b1, out_w2,
                  out_b2, ln_g, ln_b)


# speedup vs baseline: 6.0644x; 1.1560x over previous
import functools
import jax
import jax.numpy as jnp
from jax import lax
from jax.experimental import pallas as pl
from jax.experimental.pallas import tpu as pltpu
from jax.experimental.pallas import tpu_sc as plsc

N = 10000
E = 320000
DIN = 128
DOUT = 128
H = 8
HD = DOUT // H
DE = 16

NC = 2
NS = 16
NW = NC * NS
NP = 10240
NSL = NP // NS
EC = E // NW
CB = 80
NCH = EC // CB


def _erf(x):
    p = 0.3275911
    a1, a2, a3, a4, a5 = (0.254829592, -0.284496736, 1.421413741,
                          -1.453152027, 1.061405429)
    ax = jnp.abs(x)
    t = 1.0 / (1.0 + p * ax)
    poly = ((((a5 * t + a4) * t + a3) * t + a2) * t + a1) * t
    y = 1.0 - poly * jnp.exp(-ax * ax)
    return jnp.sign(x) * y


def _gelu(x):
    return 0.5 * x * (1.0 + _erf(x * 0.7071067811865476))


def _proj_body(x_ref, w_ref, o_ref):
    o_ref[...] = jnp.dot(x_ref[...], w_ref[...],
                         preferred_element_type=jnp.float32)


def _node_proj(x, w_cat):
    bn = 2000
    return pl.pallas_call(
        _proj_body,
        grid=(N // bn,),
        in_specs=[pl.BlockSpec((bn, DIN), lambda i: (i, 0)),
                  pl.BlockSpec((DIN, 4 * DOUT), lambda i: (0, 0))],
        out_specs=pl.BlockSpec((bn, 4 * DOUT), lambda i: (i, 0)),
        out_shape=jax.ShapeDtypeStruct((N, 4 * DOUT), jnp.float32),
    )(x, w_cat)


def _edge_body(s1_ref, ea_ref, w1e_ref, b1_ref, o_ref):
    m1 = s1_ref[...] + jnp.dot(
        ea_ref[...], w1e_ref[...],
        preferred_element_type=jnp.float32) + b1_ref[...]
    o_ref[...] = _gelu(m1)


def _edge_gelu(s1, ea, w1e, b1):
    be = 4000
    return pl.pallas_call(
        _edge_body,
        grid=(E // be,),
        in_specs=[pl.BlockSpec((be, DOUT), lambda i: (i, 0)),
                  pl.BlockSpec((be, DE), lambda i: (i, 0)),
                  pl.BlockSpec((DE, DOUT), lambda i: (0, 0)),
                  pl.BlockSpec((1, DOUT), lambda i: (0, 0))],
        out_specs=pl.BlockSpec((be, DOUT), lambda i: (i, 0)),
        out_shape=jax.ShapeDtypeStruct((E, DOUT), jnp.float32),
    )(s1, ea, w1e, b1.reshape(1, DOUT))


def _final_body(x_ref, agg0_ref, agg1_ref, cnt_ref, b2m_ref, w_ref, msgw2_ref,
                w1a_ref, w1b_ref, b1_ref, w2_ref, b2_ref, g_ref, bb_ref, o_ref):
    x = x_ref[...]
    aggs = agg0_ref[0] + agg1_ref[0]
    agg = jnp.dot(aggs, msgw2_ref[...], preferred_element_type=jnp.float32)
    agg = agg + jnp.dot(cnt_ref[...], b2m_ref[...],
                        preferred_element_type=jnp.float32)
    scale = jnp.repeat(w_ref[...], HD, axis=1)
    agg = agg * scale
    h = jnp.dot(x, w1a_ref[...], preferred_element_type=jnp.float32)
    h = h + jnp.dot(agg, w1b_ref[...], preferred_element_type=jnp.float32)
    h = _gelu(h + b1_ref[...])
    h = jnp.dot(h, w2_ref[...], preferred_element_type=jnp.float32) + b2_ref[...]
    h = x + h
    mu = jnp.mean(h, axis=-1, keepdims=True)
    var = jnp.mean((h - mu) ** 2, axis=-1, keepdims=True)
    o_ref[...] = (h - mu) * jax.lax.rsqrt(var + 1e-5) * g_ref[...] + bb_ref[...]


def _final(x, agg2, cnt32, w, msg_w2, msg_b2, out_w1, out_b1, out_w2, out_b2,
           ln_g, ln_b):
    bn = 2000
    row = lambda i: (i, 0)
    full = lambda i: (0, 0)
    b2m = jnp.broadcast_to(msg_b2, (NW, DOUT))
    return pl.pallas_call(
        _final_body,
        grid=(N // bn,),
        in_specs=[pl.BlockSpec((bn, DIN), row),
                  pl.BlockSpec((1, bn, DOUT), lambda i: (0, i, 0)),
                  pl.BlockSpec((1, bn, DOUT), lambda i: (1, i, 0)),
                  pl.BlockSpec((bn, NW), lambda i: (i, 0)),
                  pl.BlockSpec((NW, DOUT), full),
                  pl.BlockSpec((bn, H), row),
                  pl.BlockSpec((DOUT, DOUT), full),
                  pl.BlockSpec((DIN, DOUT), full),
                  pl.BlockSpec((DOUT, DOUT), full),
                  pl.BlockSpec((1, DOUT), full),
                  pl.BlockSpec((DOUT, DOUT), full),
                  pl.BlockSpec((1, DOUT), full),
                  pl.BlockSpec((1, DOUT), full),
                  pl.BlockSpec((1, DOUT), full)],
        out_specs=pl.BlockSpec((bn, DOUT), row),
        out_shape=jax.ShapeDtypeStruct((N, DOUT), jnp.float32),
    )(x, agg2, agg2, cnt32.T, b2m, w, msg_w2,
      out_w1[:DIN], out_w1[DIN:], out_b1.reshape(1, DOUT), out_w2,
      out_b2.reshape(1, DOUT), ln_g.reshape(1, DOUT), ln_b.reshape(1, DOUT))


_SC_MESH = plsc.VectorSubcoreMesh(core_axis_name="c", subcore_axis_name="s")


@functools.partial(
    pl.kernel,
    mesh=_SC_MESH,
    out_type=jax.ShapeDtypeStruct((E, DOUT), jnp.float32),
    scratch_types=[
        pltpu.VMEM((EC,), jnp.int32),
        pltpu.VMEM((EC,), jnp.int32),
        pltpu.VMEM((2, CB, DOUT), jnp.float32),
        pltpu.VMEM((2, CB, DOUT), jnp.float32),
        pltpu.SemaphoreType.DMA,
        pltpu.SemaphoreType.DMA,
        pltpu.SemaphoreType.DMA,
        pltpu.SemaphoreType.DMA,
    ],
    compiler_params=pltpu.CompilerParams(needs_layout_passes=False),
)
def _sc_gather(a_hbm, b_hbm, src_hbm, dst_hbm, out_s1,
               idxs_v, idxd_v, rows_a, rows_b, sem_a, sem_b, sem_w0, sem_w1):
    cid = lax.axis_index("c")
    sid = lax.axis_index("s")
    wid = sid * NC + cid
    ebase = wid * EC
    pltpu.sync_copy(src_hbm.at[pl.ds(ebase, EC)], idxs_v)
    pltpu.sync_copy(dst_hbm.at[pl.ds(ebase, EC)], idxd_v)
    sem_w = (sem_w0, sem_w1)

    def _start(i, s):
        pltpu.async_copy(a_hbm.at[idxs_v.at[pl.ds(i * CB, CB)]],
                         rows_a.at[s], sem_a)
        pltpu.async_copy(b_hbm.at[idxd_v.at[pl.ds(i * CB, CB)]],
                         rows_b.at[s], sem_b)

    def _step(i, s, prefetch):
        pltpu.make_async_copy(a_hbm.at[pl.ds(0, CB)], rows_a.at[s],
                              sem_a).wait()
        pltpu.make_async_copy(b_hbm.at[pl.ds(0, CB)], rows_b.at[s],
                              sem_b).wait()

        def _add(r, _):
            for c in range(DOUT // 16):
                sl = pl.ds(c * 16, 16)
                rows_a[s, r, sl] = rows_a[s, r, sl] + rows_b[s, r, sl]
            return 0

        lax.fori_loop(0, CB, _add, 0)
        dst_slice = out_s1.at[pl.ds(ebase + i * CB, CB)]
        pltpu.async_copy(rows_a.at[s], dst_slice, sem_w[s])
        if prefetch:
            @pl.when(i + 2 < NCH)
            def _():
                pltpu.make_async_copy(rows_a.at[s], dst_slice,
                                      sem_w[s]).wait()
                _start(i + 2, s)

    _start(0, 0)
    _start(1, 1)

    def _pair(i2, _):
        _step(2 * i2, 0, True)
        _step(2 * i2 + 1, 1, True)
        return 0

    lax.fori_loop(0, NCH // 2, _pair, 0)
    _step(NCH - 1, 0, False)
    pltpu.make_async_copy(rows_a.at[0], out_s1.at[pl.ds(0, CB)], sem_w0).wait()
    pltpu.make_async_copy(rows_a.at[1], out_s1.at[pl.ds(0, CB)], sem_w1).wait()


@functools.partial(
    pl.kernel,
    mesh=_SC_MESH,
    out_type=[
        jax.ShapeDtypeStruct((NC, NP, DOUT), jnp.float32),
        jax.ShapeDtypeStruct((NW, NP), jnp.float32),
        jax.ShapeDtypeStruct((NW * NP,), jnp.int32),
    ],
    scratch_types=[
        pltpu.VMEM((2, CB), jnp.int32),
        pltpu.VMEM((2, CB, DOUT), jnp.float32),
        pltpu.VMEM((NP,), jnp.float32),
        pltpu.VMEM((NP,), jnp.int32),
        pltpu.VMEM_SHARED((NP, DOUT), jnp.float32),
        pltpu.SemaphoreType.DMA,
        pltpu.SemaphoreType.DMA,
    ],
    compiler_params=pltpu.CompilerParams(needs_layout_passes=False),
)
def _sc_aggregate(g_hbm, dst_hbm, zero_hbm, out_agg, out_cnt, out_win,
                  idx_v, rows_v, cnt_v, win_v, agg_sh, sem_g, sem_i):
    cid = lax.axis_index("c")
    sid = lax.axis_index("s")
    wid = sid * NC + cid
    lane = lax.iota(jnp.int32, 16)
    ones = jnp.ones((16,), jnp.float32)
    lane_masks = [lane == l for l in range(16)]

    def _init(i, _):
        cnt_v[pl.ds(i * 16, 16)] = jnp.zeros((16,), jnp.float32)
        win_v[pl.ds(i * 16, 16)] = jnp.full((16,), -1, jnp.int32)
        return 0

    lax.fori_loop(0, NP // 16, _init, 0)

    pltpu.sync_copy(zero_hbm, agg_sh.at[pl.ds(sid * NSL, NSL)])
    plsc.subcore_barrier()
    ebase = wid * EC

    def _startg(i, s):
        pltpu.async_copy(dst_hbm.at[pl.ds(ebase + i * CB, CB)], idx_v.at[s],
                         sem_i)
        pltpu.async_copy(g_hbm.at[pl.ds(ebase + i * CB, CB)], rows_v.at[s],
                         sem_g)

    def _step(i, s, prefetch):
        pltpu.make_async_copy(dst_hbm.at[pl.ds(0, CB)], idx_v.at[s],
                              sem_i).wait()
        pltpu.make_async_copy(g_hbm.at[pl.ds(0, CB)], rows_v.at[s],
                              sem_g).wait()
        pltpu.sync_copy(rows_v.at[s], agg_sh.at[idx_v.at[s]], add=True)
        for j in range(CB // 16):
            d16 = idx_v[s, pl.ds(j * 16, 16)]
            plsc.addupdate_scatter(cnt_v, [d16], ones)
            e16 = lane + (ebase + i * CB + j * 16)
            for l in range(16):
                plsc.store_scatter(win_v, [d16], e16, mask=lane_masks[l])
        if prefetch:
            @pl.when(i + 2 < NCH)
            def _():
                _startg(i + 2, s)

    _startg(0, 0)
    _startg(1, 1)

    def _pair(i2, _):
        _step(2 * i2, 0, True)
        _step(2 * i2 + 1, 1, True)
        return 0

    lax.fori_loop(0, NCH // 2, _pair, 0)
    _step(NCH - 1, 0, False)
    plsc.subcore_barrier()

    pltpu.sync_copy(win_v, out_win.at[pl.ds(wid * NP, NP)])
    pltpu.sync_copy(cnt_v, out_cnt.at[wid])
    nbase = sid * NSL
    pltpu.sync_copy(agg_sh.at[pl.ds(nbase, NSL)], out_agg.at[cid, pl.ds(nbase, NSL)])


NPW = NP // NW
WGB = 80


E8 = E // 8


@functools.partial(
    pl.kernel,
    mesh=_SC_MESH,
    out_type=[
        jax.ShapeDtypeStruct((NP, DOUT), jnp.float32),
        jax.ShapeDtypeStruct((NP, 128), jnp.float32),
        jax.ShapeDtypeStruct((NP,), jnp.int32),
    ],
    scratch_types=[
        pltpu.VMEM((NPW,), jnp.int32),
        pltpu.VMEM((NPW,), jnp.int32),
        pltpu.VMEM((NPW,), jnp.int32),
        pltpu.VMEM((NPW,), jnp.int32),
        pltpu.VMEM((WGB,), jnp.int32),
        pltpu.VMEM((WGB, 128), jnp.int32),
        pltpu.VMEM((WGB, DOUT), jnp.float32),
        pltpu.VMEM((WGB, 128), jnp.float32),
        pltpu.SemaphoreType.DMA,
    ],
    compiler_params=pltpu.CompilerParams(needs_layout_passes=False),
)
def _sc_winner_gather(win32_hbm, srcp_hbm, ea8_hbm, k_hbm,
                      out_k, out_ea, out_win,
                      wa_v, wb_v, wc_v, wd_v, sw_v, srcrows_v, krows_v,
                      earows_v, sem):
    cid = lax.axis_index("c")
    sid = lax.axis_index("s")
    wid = sid * NC + cid
    nbase = wid * NPW
    lane = lax.iota(jnp.int32, 16)
    pltpu.sync_copy(win32_hbm.at[pl.ds(nbase, NPW)], wa_v)
    for t in range(1, NW):
        pltpu.sync_copy(win32_hbm.at[pl.ds(t * NP + nbase, NPW)], wb_v)

        def _m1(k, _):
            wa_v[pl.ds(k * 16, 16)] = jnp.maximum(
                wa_v[pl.ds(k * 16, 16)], wb_v[pl.ds(k * 16, 16)])
            return 0

        lax.fori_loop(0, NPW // 16, _m1, 0)

    def _mx(k, _):
        m = wa_v[pl.ds(k * 16, 16)]
        c = jnp.maximum(m, 0)
        wc_v[pl.ds(k * 16, 16)] = c
        wd_v[pl.ds(k * 16, 16)] = c >> 3
        return 0

    lax.fori_loop(0, NPW // 16, _mx, 0)
    pltpu.sync_copy(wa_v, out_win.at[pl.ds(nbase, NPW)])
    for k in range(NPW // WGB):
        idx8 = wd_v.at[pl.ds(k * WGB, WGB)]
        pltpu.async_copy(srcp_hbm.at[idx8], srcrows_v, sem).wait()
        pltpu.async_copy(ea8_hbm.at[idx8], earows_v, sem).wait()
        for t in range(WGB // 16):
            cols = wc_v[pl.ds(k * WGB + t * 16, 16)] & 7
            rows = lane + (t * 16)
            sw_v[pl.ds(t * 16, 16)] = plsc.load_gather(srcrows_v, [rows, cols])
        pltpu.async_copy(k_hbm.at[sw_v], krows_v, sem).wait()
        pltpu.sync_copy(earows_v, out_ea.at[pl.ds(nbase + k * WGB, WGB)])
        pltpu.sync_copy(krows_v, out_k.at[pl.ds(nbase + k * WGB, WGB)])


def _attn_body(q_ref, qb_ref, kw_ref, ea_ref, win_ref, kvw_ref, kvb_ref,
               seg_ref, o_ref):
    q = q_ref[...] + qb_ref[...]
    sub = jax.lax.broadcasted_iota(jnp.int32, (N, 128), 1) // DE
    ea = jnp.where(sub == (win_ref[...] & 7), ea_ref[...], 0.0)
    kv = kw_ref[...] + jnp.dot(ea, kvw_ref[...],
                               preferred_element_type=jnp.float32) + kvb_ref[...]
    prod = q * kv
    attn = jnp.dot(prod, seg_ref[...], preferred_element_type=jnp.float32)
    attn = attn * (HD ** -0.5)
    valid = win_ref[...] >= 0
    attn = jnp.where(valid, attn, -jnp.inf)
    mx = jnp.max(attn, axis=0, keepdims=True)
    ex = jnp.where(valid, jnp.exp(attn - mx), 0.0)
    o_ref[...] = ex / jnp.sum(ex, axis=0, keepdims=True)


def _attention(q_raw, q_b, kw, eaw, winm, kv_we, kv_b, seg):
    full = lambda: None
    return pl.pallas_call(
        _attn_body,
        grid=(1,),
        in_specs=[pl.BlockSpec((N, DOUT), lambda i: (0, 0)),
                  pl.BlockSpec((1, DOUT), lambda i: (0, 0)),
                  pl.BlockSpec((N, DOUT), lambda i: (0, 0)),
                  pl.BlockSpec((N, 128), lambda i: (0, 0)),
                  pl.BlockSpec((N, 1), lambda i: (0, 0)),
                  pl.BlockSpec((128, DOUT), lambda i: (0, 0)),
                  pl.BlockSpec((1, DOUT), lambda i: (0, 0)),
                  pl.BlockSpec((DOUT, H), lambda i: (0, 0))],
        out_specs=pl.BlockSpec((N, H), lambda i: (0, 0)),
        out_shape=jax.ShapeDtypeStruct((N, H), jnp.float32),
    )(q_raw, q_b.reshape(1, DOUT), kw, eaw, winm.reshape(N, 1), kv_we,
      kv_b.reshape(1, DOUT), seg)


def kernel(node_features, edge_index, edge_attr, msg_w1, msg_b1, msg_w2, msg_b2,
           q_w, q_b, kv_w, kv_b, out_w1, out_b1, out_w2, out_b2, ln_g, ln_b):
    x = node_features
    src, dst = edge_index[0], edge_index[1]

    w_cat = jnp.concatenate([msg_w1[:DIN], msg_w1[DIN:2 * DIN], q_w,
                             kv_w[:DIN]], axis=1)
    P = _node_proj(x, w_cat)
    A, B, Qr, K = (P[:, :DOUT], P[:, DOUT:2 * DOUT],
                   P[:, 2 * DOUT:3 * DOUT], P[:, 3 * DOUT:])

    s1 = _sc_gather(A, B, src, dst)
    g = _edge_gelu(s1, edge_attr, msg_w1[2 * DIN:], msg_b1)

    zero_slab = jnp.zeros((NSL, DOUT), jnp.float32)
    agg2, cnt32, win32 = _sc_aggregate(g, dst, zero_slab)

    srcp = jnp.pad(src.reshape(E8, 8), ((0, 0), (0, 120)))
    ea8 = edge_attr.reshape(E8, 128)
    kw, eaw8, winm = _sc_winner_gather(win32, srcp, ea8, K)
    seg = jnp.repeat(jnp.eye(H, dtype=jnp.float32), HD, axis=0)
    kv128 = jnp.tile(kv_w[DIN:], (8, 1))
    w = _attention(Qr, q_b, kw[:N], eaw8[:N], winm[:N], kv128, kv_b, seg)

    return _final(x, agg2, cnt32, w, msg_w2, msg_b2, out_w1, out_b1, out_w2,
                  out_b2, ln_g, ln_b)


# speedup vs baseline: 6.1419x; 1.0128x over previous
import functools
import jax
import jax.numpy as jnp
from jax import lax
from jax.experimental import pallas as pl
from jax.experimental.pallas import tpu as pltpu
from jax.experimental.pallas import tpu_sc as plsc

N = 10000
E = 320000
DIN = 128
DOUT = 128
H = 8
HD = DOUT // H
DE = 16

NC = 2
NS = 16
NW = NC * NS
NP = 10240
NSL = NP // NS
EC = E // NW
CB = 80
NCH = EC // CB


def _erf(x):
    p = 0.3275911
    a1, a2, a3, a4, a5 = (0.254829592, -0.284496736, 1.421413741,
                          -1.453152027, 1.061405429)
    ax = jnp.abs(x)
    t = 1.0 / (1.0 + p * ax)
    poly = ((((a5 * t + a4) * t + a3) * t + a2) * t + a1) * t
    y = 1.0 - poly * jnp.exp(-ax * ax)
    return jnp.sign(x) * y


def _gelu(x):
    return 0.5 * x * (1.0 + _erf(x * 0.7071067811865476))


def _proj_body(x_ref, w_ref, o_ref):
    o_ref[...] = jnp.dot(x_ref[...], w_ref[...],
                         preferred_element_type=jnp.float32)


def _node_proj(x, w_cat):
    bn = 2000
    return pl.pallas_call(
        _proj_body,
        grid=(N // bn,),
        in_specs=[pl.BlockSpec((bn, DIN), lambda i: (i, 0)),
                  pl.BlockSpec((DIN, 4 * DOUT), lambda i: (0, 0))],
        out_specs=pl.BlockSpec((bn, 4 * DOUT), lambda i: (i, 0)),
        out_shape=jax.ShapeDtypeStruct((N, 4 * DOUT), jnp.float32),
    )(x, w_cat)


def _edge_body(s1_ref, ea_ref, w1e_ref, b1_ref, o_ref):
    m1 = s1_ref[...] + jnp.dot(
        ea_ref[...], w1e_ref[...],
        preferred_element_type=jnp.float32) + b1_ref[...]
    o_ref[...] = _gelu(m1)


def _edge_gelu(s1, ea, w1e, b1):
    be = 4000
    return pl.pallas_call(
        _edge_body,
        grid=(E // be,),
        in_specs=[pl.BlockSpec((be, DOUT), lambda i: (i, 0)),
                  pl.BlockSpec((be, DE), lambda i: (i, 0)),
                  pl.BlockSpec((DE, DOUT), lambda i: (0, 0)),
                  pl.BlockSpec((1, DOUT), lambda i: (0, 0))],
        out_specs=pl.BlockSpec((be, DOUT), lambda i: (i, 0)),
        out_shape=jax.ShapeDtypeStruct((E, DOUT), jnp.float32),
    )(s1, ea, w1e, b1.reshape(1, DOUT))


def _final_body(x_ref, agg0_ref, agg1_ref, cnt_ref, b2m_ref, w_ref, msgw2_ref,
                w1a_ref, w1b_ref, b1_ref, w2_ref, b2_ref, g_ref, bb_ref, o_ref):
    x = x_ref[...]
    aggs = agg0_ref[0] + agg1_ref[0]
    agg = jnp.dot(aggs, msgw2_ref[...], preferred_element_type=jnp.float32)
    agg = agg + jnp.dot(cnt_ref[...], b2m_ref[...],
                        preferred_element_type=jnp.float32)
    scale = jnp.repeat(w_ref[...], HD, axis=1)
    agg = agg * scale
    h = jnp.dot(x, w1a_ref[...], preferred_element_type=jnp.float32)
    h = h + jnp.dot(agg, w1b_ref[...], preferred_element_type=jnp.float32)
    h = _gelu(h + b1_ref[...])
    h = jnp.dot(h, w2_ref[...], preferred_element_type=jnp.float32) + b2_ref[...]
    h = x + h
    mu = jnp.mean(h, axis=-1, keepdims=True)
    var = jnp.mean((h - mu) ** 2, axis=-1, keepdims=True)
    o_ref[...] = (h - mu) * jax.lax.rsqrt(var + 1e-5) * g_ref[...] + bb_ref[...]


def _final(x, agg2, cnt32, w, msg_w2, msg_b2, out_w1, out_b1, out_w2, out_b2,
           ln_g, ln_b):
    bn = 2000
    row = lambda i: (i, 0)
    full = lambda i: (0, 0)
    b2m = jnp.broadcast_to(msg_b2, (NW, DOUT))
    return pl.pallas_call(
        _final_body,
        grid=(N // bn,),
        in_specs=[pl.BlockSpec((bn, DIN), row),
                  pl.BlockSpec((1, bn, DOUT), lambda i: (0, i, 0)),
                  pl.BlockSpec((1, bn, DOUT), lambda i: (1, i, 0)),
                  pl.BlockSpec((bn, NW), lambda i: (i, 0)),
                  pl.BlockSpec((NW, DOUT), full),
                  pl.BlockSpec((bn, H), row),
                  pl.BlockSpec((DOUT, DOUT), full),
                  pl.BlockSpec((DIN, DOUT), full),
                  pl.BlockSpec((DOUT, DOUT), full),
                  pl.BlockSpec((1, DOUT), full),
                  pl.BlockSpec((DOUT, DOUT), full),
                  pl.BlockSpec((1, DOUT), full),
                  pl.BlockSpec((1, DOUT), full),
                  pl.BlockSpec((1, DOUT), full)],
        out_specs=pl.BlockSpec((bn, DOUT), row),
        out_shape=jax.ShapeDtypeStruct((N, DOUT), jnp.float32),
    )(x, agg2, agg2, cnt32.T, b2m, w, msg_w2,
      out_w1[:DIN], out_w1[DIN:], out_b1.reshape(1, DOUT), out_w2,
      out_b2.reshape(1, DOUT), ln_g.reshape(1, DOUT), ln_b.reshape(1, DOUT))


_SC_MESH = plsc.VectorSubcoreMesh(core_axis_name="c", subcore_axis_name="s")


@functools.partial(
    pl.kernel,
    mesh=_SC_MESH,
    out_type=[
        jax.ShapeDtypeStruct((E, DOUT), jnp.float32),
        jax.ShapeDtypeStruct((NW, NP), jnp.float32),
        jax.ShapeDtypeStruct((NW * NP,), jnp.int32),
    ],
    scratch_types=[
        pltpu.VMEM((EC,), jnp.int32),
        pltpu.VMEM((EC,), jnp.int32),
        pltpu.VMEM((2, CB, DOUT), jnp.float32),
        pltpu.VMEM((2, CB, DOUT), jnp.float32),
        pltpu.VMEM((NP,), jnp.float32),
        pltpu.VMEM((NP,), jnp.int32),
        pltpu.SemaphoreType.DMA,
        pltpu.SemaphoreType.DMA,
        pltpu.SemaphoreType.DMA,
        pltpu.SemaphoreType.DMA,
    ],
    compiler_params=pltpu.CompilerParams(needs_layout_passes=False),
)
def _sc_gather(a_hbm, b_hbm, src_hbm, dst_hbm, out_s1, out_cnt, out_win,
               idxs_v, idxd_v, rows_a, rows_b, cnt_v, win_v,
               sem_a, sem_b, sem_w0, sem_w1):
    cid = lax.axis_index("c")
    sid = lax.axis_index("s")
    wid = sid * NC + cid
    ebase = wid * EC
    lane = lax.iota(jnp.int32, 16)
    ones = jnp.ones((16,), jnp.float32)
    lane_masks = [lane == l for l in range(16)]
    pltpu.sync_copy(src_hbm.at[pl.ds(ebase, EC)], idxs_v)
    pltpu.sync_copy(dst_hbm.at[pl.ds(ebase, EC)], idxd_v)
    sem_w = (sem_w0, sem_w1)

    def _init(i, _):
        cnt_v[pl.ds(i * 16, 16)] = jnp.zeros((16,), jnp.float32)
        win_v[pl.ds(i * 16, 16)] = jnp.full((16,), -1, jnp.int32)
        return 0

    lax.fori_loop(0, NP // 16, _init, 0)

    def _start(i, s):
        pltpu.async_copy(a_hbm.at[idxs_v.at[pl.ds(i * CB, CB)]],
                         rows_a.at[s], sem_a)
        pltpu.async_copy(b_hbm.at[idxd_v.at[pl.ds(i * CB, CB)]],
                         rows_b.at[s], sem_b)

    def _step(i, s, prefetch):
        pltpu.make_async_copy(a_hbm.at[pl.ds(0, CB)], rows_a.at[s],
                              sem_a).wait()
        pltpu.make_async_copy(b_hbm.at[pl.ds(0, CB)], rows_b.at[s],
                              sem_b).wait()

        def _add(r, _):
            for c in range(DOUT // 16):
                sl = pl.ds(c * 16, 16)
                rows_a[s, r, sl] = rows_a[s, r, sl] + rows_b[s, r, sl]
            return 0

        lax.fori_loop(0, CB, _add, 0)
        dst_slice = out_s1.at[pl.ds(ebase + i * CB, CB)]
        pltpu.async_copy(rows_a.at[s], dst_slice, sem_w[s])
        for j in range(CB // 16):
            d16 = idxd_v[pl.ds(i * CB + j * 16, 16)]
            plsc.addupdate_scatter(cnt_v, [d16], ones)
            e16 = lane + (ebase + i * CB + j * 16)
            for l in range(16):
                plsc.store_scatter(win_v, [d16], e16, mask=lane_masks[l])
        if prefetch:
            @pl.when(i + 2 < NCH)
            def _():
                pltpu.make_async_copy(rows_a.at[s], dst_slice,
                                      sem_w[s]).wait()
                _start(i + 2, s)

    _start(0, 0)
    _start(1, 1)

    def _pair(i2, _):
        _step(2 * i2, 0, True)
        _step(2 * i2 + 1, 1, True)
        return 0

    lax.fori_loop(0, NCH // 2, _pair, 0)
    _step(NCH - 1, 0, False)
    pltpu.sync_copy(cnt_v, out_cnt.at[wid])
    pltpu.sync_copy(win_v, out_win.at[pl.ds(wid * NP, NP)])
    pltpu.make_async_copy(rows_a.at[0], out_s1.at[pl.ds(0, CB)], sem_w0).wait()
    pltpu.make_async_copy(rows_a.at[1], out_s1.at[pl.ds(0, CB)], sem_w1).wait()


@functools.partial(
    pl.kernel,
    mesh=_SC_MESH,
    out_type=jax.ShapeDtypeStruct((NC, NP, DOUT), jnp.float32),
    scratch_types=[
        pltpu.VMEM((2, CB), jnp.int32),
        pltpu.VMEM((2, CB, DOUT), jnp.float32),
        pltpu.VMEM_SHARED((NP, DOUT), jnp.float32),
        pltpu.SemaphoreType.DMA,
        pltpu.SemaphoreType.DMA,
    ],
    compiler_params=pltpu.CompilerParams(needs_layout_passes=False),
)
def _sc_aggregate(g_hbm, dst_hbm, zero_hbm, out_agg,
                  idx_v, rows_v, agg_sh, sem_g, sem_i):
    cid = lax.axis_index("c")
    sid = lax.axis_index("s")
    wid = sid * NC + cid

    pltpu.sync_copy(zero_hbm, agg_sh.at[pl.ds(sid * NSL, NSL)])
    plsc.subcore_barrier()
    ebase = wid * EC

    def _startg(i, s):
        pltpu.async_copy(dst_hbm.at[pl.ds(ebase + i * CB, CB)], idx_v.at[s],
                         sem_i)
        pltpu.async_copy(g_hbm.at[pl.ds(ebase + i * CB, CB)], rows_v.at[s],
                         sem_g)

    def _step(i, s, prefetch):
        pltpu.make_async_copy(dst_hbm.at[pl.ds(0, CB)], idx_v.at[s],
                              sem_i).wait()
        pltpu.make_async_copy(g_hbm.at[pl.ds(0, CB)], rows_v.at[s],
                              sem_g).wait()
        pltpu.sync_copy(rows_v.at[s], agg_sh.at[idx_v.at[s]], add=True)
        if prefetch:
            @pl.when(i + 2 < NCH)
            def _():
                _startg(i + 2, s)

    _startg(0, 0)
    _startg(1, 1)

    def _pair(i2, _):
        _step(2 * i2, 0, True)
        _step(2 * i2 + 1, 1, True)
        return 0

    lax.fori_loop(0, NCH // 2, _pair, 0)
    _step(NCH - 1, 0, False)
    plsc.subcore_barrier()

    nbase = sid * NSL
    pltpu.sync_copy(agg_sh.at[pl.ds(nbase, NSL)], out_agg.at[cid, pl.ds(nbase, NSL)])


NPW = NP // NW
WGB = 80


E8 = E // 8


@functools.partial(
    pl.kernel,
    mesh=_SC_MESH,
    out_type=[
        jax.ShapeDtypeStruct((NP, DOUT), jnp.float32),
        jax.ShapeDtypeStruct((NP, 128), jnp.float32),
        jax.ShapeDtypeStruct((NP,), jnp.int32),
    ],
    scratch_types=[
        pltpu.VMEM((NPW,), jnp.int32),
        pltpu.VMEM((NPW,), jnp.int32),
        pltpu.VMEM((NPW,), jnp.int32),
        pltpu.VMEM((NPW,), jnp.int32),
        pltpu.VMEM((WGB,), jnp.int32),
        pltpu.VMEM((WGB, 128), jnp.int32),
        pltpu.VMEM((WGB, DOUT), jnp.float32),
        pltpu.VMEM((WGB, 128), jnp.float32),
        pltpu.SemaphoreType.DMA,
    ],
    compiler_params=pltpu.CompilerParams(needs_layout_passes=False),
)
def _sc_winner_gather(win32_hbm, srcp_hbm, ea8_hbm, k_hbm,
                      out_k, out_ea, out_win,
                      wa_v, wb_v, wc_v, wd_v, sw_v, srcrows_v, krows_v,
                      earows_v, sem):
    cid = lax.axis_index("c")
    sid = lax.axis_index("s")
    wid = sid * NC + cid
    nbase = wid * NPW
    lane = lax.iota(jnp.int32, 16)
    pltpu.sync_copy(win32_hbm.at[pl.ds(nbase, NPW)], wa_v)
    for t in range(1, NW):
        pltpu.sync_copy(win32_hbm.at[pl.ds(t * NP + nbase, NPW)], wb_v)

        def _m1(k, _):
            wa_v[pl.ds(k * 16, 16)] = jnp.maximum(
                wa_v[pl.ds(k * 16, 16)], wb_v[pl.ds(k * 16, 16)])
            return 0

        lax.fori_loop(0, NPW // 16, _m1, 0)

    def _mx(k, _):
        m = wa_v[pl.ds(k * 16, 16)]
        c = jnp.maximum(m, 0)
        wc_v[pl.ds(k * 16, 16)] = c
        wd_v[pl.ds(k * 16, 16)] = c >> 3
        return 0

    lax.fori_loop(0, NPW // 16, _mx, 0)
    pltpu.sync_copy(wa_v, out_win.at[pl.ds(nbase, NPW)])
    for k in range(NPW // WGB):
        idx8 = wd_v.at[pl.ds(k * WGB, WGB)]
        pltpu.async_copy(srcp_hbm.at[idx8], srcrows_v, sem).wait()
        pltpu.async_copy(ea8_hbm.at[idx8], earows_v, sem).wait()
        for t in range(WGB // 16):
            cols = wc_v[pl.ds(k * WGB + t * 16, 16)] & 7
            rows = lane + (t * 16)
            sw_v[pl.ds(t * 16, 16)] = plsc.load_gather(srcrows_v, [rows, cols])
        pltpu.async_copy(k_hbm.at[sw_v], krows_v, sem).wait()
        pltpu.sync_copy(earows_v, out_ea.at[pl.ds(nbase + k * WGB, WGB)])
        pltpu.sync_copy(krows_v, out_k.at[pl.ds(nbase + k * WGB, WGB)])


def _attn_body(q_ref, qb_ref, kw_ref, ea_ref, win_ref, kvw_ref, kvb_ref,
               seg_ref, o_ref):
    q = q_ref[...] + qb_ref[...]
    sub = jax.lax.broadcasted_iota(jnp.int32, (N, 128), 1) // DE
    ea = jnp.where(sub == (win_ref[...] & 7), ea_ref[...], 0.0)
    kv = kw_ref[...] + jnp.dot(ea, kvw_ref[...],
                               preferred_element_type=jnp.float32) + kvb_ref[...]
    prod = q * kv
    attn = jnp.dot(prod, seg_ref[...], preferred_element_type=jnp.float32)
    attn = attn * (HD ** -0.5)
    valid = win_ref[...] >= 0
    attn = jnp.where(valid, attn, -jnp.inf)
    mx = jnp.max(attn, axis=0, keepdims=True)
    ex = jnp.where(valid, jnp.exp(attn - mx), 0.0)
    o_ref[...] = ex / jnp.sum(ex, axis=0, keepdims=True)


def _attention(q_raw, q_b, kw, eaw, winm, kv_we, kv_b, seg):
    full = lambda: None
    return pl.pallas_call(
        _attn_body,
        grid=(1,),
        in_specs=[pl.BlockSpec((N, DOUT), lambda i: (0, 0)),
                  pl.BlockSpec((1, DOUT), lambda i: (0, 0)),
                  pl.BlockSpec((N, DOUT), lambda i: (0, 0)),
                  pl.BlockSpec((N, 128), lambda i: (0, 0)),
                  pl.BlockSpec((N, 1), lambda i: (0, 0)),
                  pl.BlockSpec((128, DOUT), lambda i: (0, 0)),
                  pl.BlockSpec((1, DOUT), lambda i: (0, 0)),
                  pl.BlockSpec((DOUT, H), lambda i: (0, 0))],
        out_specs=pl.BlockSpec((N, H), lambda i: (0, 0)),
        out_shape=jax.ShapeDtypeStruct((N, H), jnp.float32),
    )(q_raw, q_b.reshape(1, DOUT), kw, eaw, winm.reshape(N, 1), kv_we,
      kv_b.reshape(1, DOUT), seg)


def kernel(node_features, edge_index, edge_attr, msg_w1, msg_b1, msg_w2, msg_b2,
           q_w, q_b, kv_w, kv_b, out_w1, out_b1, out_w2, out_b2, ln_g, ln_b):
    x = node_features
    src, dst = edge_index[0], edge_index[1]

    w_cat = jnp.concatenate([msg_w1[:DIN], msg_w1[DIN:2 * DIN], q_w,
                             kv_w[:DIN]], axis=1)
    P = _node_proj(x, w_cat)
    A, B, Qr, K = (P[:, :DOUT], P[:, DOUT:2 * DOUT],
                   P[:, 2 * DOUT:3 * DOUT], P[:, 3 * DOUT:])

    s1, cnt32, win32 = _sc_gather(A, B, src, dst)
    g = _edge_gelu(s1, edge_attr, msg_w1[2 * DIN:], msg_b1)

    zero_slab = jnp.zeros((NSL, DOUT), jnp.float32)
    agg2 = _sc_aggregate(g, dst, zero_slab)

    srcp = jnp.pad(src.reshape(E8, 8), ((0, 0), (0, 120)))
    ea8 = edge_attr.reshape(E8, 128)
    kw, eaw8, winm = _sc_winner_gather(win32, srcp, ea8, K)
    seg = jnp.repeat(jnp.eye(H, dtype=jnp.float32), HD, axis=0)
    kv128 = jnp.tile(kv_w[DIN:], (8, 1))
    w = _attention(Qr, q_b, kw[:N], eaw8[:N], winm[:N], kv128, kv_b, seg)

    return _final(x, agg2, cnt32, w, msg_w2, msg_b2, out_w1, out_b1, out_w2,
                  out_b2, ln_g, ln_b)
